# Initial kernel scaffold; baseline (speedup 1.0000x reference)
import jax
import jax.numpy as jnp
from jax.experimental import pallas as pl


def kernel(x, Wr, W1, b1, W2, b2):
    raise NotImplementedError("write your pallas kernel here")


# speedup vs baseline: 5.4324x; 5.4324x over previous
import functools

import jax
import jax.numpy as jnp
from jax import lax
from jax.experimental import pallas as pl
from jax.experimental.pallas import tpu as pltpu
from jax.experimental.pallas import tpu_sc as plsc

B, S, DIM, HID, E, K = 2, 2048, 1024, 4096, 8, 2
N = B * S
A = N * K
T = 128
M = A + E * T
NT = M // T
TB = 512
NTR = N // TB
E16 = 16

NC, NS = 2, 16
NW = NC * NS
TOK_W = N // NW
CH = 32
CH2 = 16


def _router_body(x_ref, wr_ref, e0_ref, e1_ref, r0_ref, r1_ref,
                 g0_ref, g1_ref, cnt_ref, aux_ref, cnt_acc, psum_acc):
    i = pl.program_id(0)
    x = x_ref[...]
    logits = jnp.dot(x, wr_ref[...], preferred_element_type=jnp.float32)
    m = jnp.max(logits, axis=-1, keepdims=True)
    p = jnp.exp(logits - m)
    p = p / jnp.sum(p, axis=-1, keepdims=True)

    i0 = jnp.argmax(p, axis=-1)
    eidx = lax.broadcasted_iota(jnp.int32, (TB, E16), 1)
    oh0 = (eidx == i0[:, None]).astype(jnp.float32)
    pm = jnp.where(eidx[:, :E] == i0[:, None], -jnp.inf, p)
    i1 = jnp.argmax(pm, axis=-1)
    oh1 = (eidx == i1[:, None]).astype(jnp.float32)
    m0 = jnp.max(p, axis=-1)
    m1 = jnp.max(pm, axis=-1)
    den = m0 + m1 + 1e-9
    g0 = m0 / den
    g1 = m1 / den

    @pl.when(i == 0)
    def _():
        cnt_acc[...] = jnp.zeros_like(cnt_acc)
        psum_acc[...] = jnp.zeros_like(psum_acc)

    ohsum = oh0 + oh1
    rows = lax.broadcasted_iota(jnp.int32, (TB, TB), 0)
    cols = lax.broadcasted_iota(jnp.int32, (TB, TB), 1)
    ltri = (cols < rows).astype(jnp.float32)
    csum = jnp.dot(ltri, ohsum, preferred_element_type=jnp.float32)
    before = cnt_acc[...] + csum
    r0 = jnp.sum(before * oh0, axis=-1)
    r1 = jnp.sum(before * oh1, axis=-1)

    e0_ref[...] = i0.astype(jnp.int32)[:, None]
    e1_ref[...] = i1.astype(jnp.int32)[:, None]
    r0_ref[...] = r0.astype(jnp.int32)[:, None]
    r1_ref[...] = r1.astype(jnp.int32)[:, None]
    g0_ref[...] = g0[:, None]
    g1_ref[...] = g1[:, None]

    newcnt = cnt_acc[...] + jnp.sum(ohsum, axis=0, keepdims=True)
    cnt_acc[...] = newcnt
    newpsum = psum_acc[...] + jnp.sum(p, axis=0, keepdims=True)
    psum_acc[...] = newpsum

    @pl.when(i == NTR - 1)
    def _():
        cnt_ref[...] = newcnt.astype(jnp.int32)
        avg_tokens = newcnt[0, :E] / float(N * K)
        avg_prob = newpsum[0] / float(N)
        aux_ref[...] = (0.01 * E * jnp.sum(avg_tokens * avg_prob,
                                           keepdims=True))[:, None]


def _router(x2, Wr):
    iv = [jax.ShapeDtypeStruct((N, 1), jnp.int32)] * 4
    fv = [jax.ShapeDtypeStruct((N, 1), jnp.float32)] * 2
    return pl.pallas_call(
        _router_body,
        grid=(NTR,),
        in_specs=[
            pl.BlockSpec((TB, DIM), lambda i: (i, 0)),
            pl.BlockSpec((DIM, E), lambda i: (0, 0)),
        ],
        out_specs=[pl.BlockSpec((TB, 1), lambda i: (i, 0))] * 6
        + [pl.BlockSpec((1, E16), lambda i: (0, 0)),
           pl.BlockSpec((1, 1), lambda i: (0, 0))],
        out_shape=iv + fv + [jax.ShapeDtypeStruct((1, E16), jnp.int32),
                             jax.ShapeDtypeStruct((1, 1), jnp.float32)],
        scratch_shapes=[pltpu.VMEM((1, E16), jnp.float32),
                        pltpu.VMEM((1, E), jnp.float32)],
        compiler_params=pltpu.CompilerParams(
            dimension_semantics=("arbitrary",)),
    )(x2, Wr)


def _take16(vec, idx):
    return lax.gather(
        vec, idx[:, None],
        dimension_numbers=lax.GatherDimensionNumbers(
            offset_dims=(), collapsed_slice_dims=(0,), start_index_map=(0,)),
        slice_sizes=(1,),
        mode=lax.GatherScatterMode.PROMISE_IN_BOUNDS)

def _dispatch_body(x_hbm, e0_hbm, e1_hbm, r0_hbm, r1_hbm, base_hbm,
              xs_hbm, pos0_hbm, pos1_hbm,
              xbuf, basev, e0v, e1v, r0v, r1v, p0v, p1v, sem):
    wid = lax.axis_index("s") * NC + lax.axis_index("c")
    t0 = wid * TOK_W
    pltpu.sync_copy(base_hbm, basev)
    basereg = basev[...]

    def chunk(ci, carry):
        off = t0 + ci * CH
        pltpu.sync_copy(e0_hbm.at[pl.ds(off, CH)], e0v)
        pltpu.sync_copy(e1_hbm.at[pl.ds(off, CH)], e1v)
        pltpu.sync_copy(r0_hbm.at[pl.ds(off, CH)], r0v)
        pltpu.sync_copy(r1_hbm.at[pl.ds(off, CH)], r1v)
        for j in range(CH // 16):
            sl = pl.ds(j * 16, 16)
            p0v[sl] = _take16(basereg, e0v[sl]) + r0v[sl]
            p1v[sl] = _take16(basereg, e1v[sl]) + r1v[sl]
        pltpu.sync_copy(x_hbm.at[pl.ds(off, CH)], xbuf)
        pltpu.async_copy(xbuf, xs_hbm.at[p0v], sem).wait()
        pltpu.async_copy(xbuf, xs_hbm.at[p1v], sem).wait()
        pltpu.sync_copy(p0v, pos0_hbm.at[pl.ds(off, CH)])
        pltpu.sync_copy(p1v, pos1_hbm.at[pl.ds(off, CH)])
        return carry

    lax.fori_loop(0, TOK_W // CH, chunk, 0)


@functools.lru_cache(maxsize=None)
def _make_dispatch():
    mesh = plsc.VectorSubcoreMesh(core_axis_name="c", subcore_axis_name="s")
    return pl.kernel(
        _dispatch_body, mesh=mesh,
        out_type=[
            jax.ShapeDtypeStruct((M, DIM), jnp.float32),
            jax.ShapeDtypeStruct((N,), jnp.int32),
            jax.ShapeDtypeStruct((N,), jnp.int32),
        ],
        scratch_types=[
            pltpu.VMEM((CH, DIM), jnp.float32),
            pltpu.VMEM((E16,), jnp.int32),
            pltpu.VMEM((CH,), jnp.int32),
            pltpu.VMEM((CH,), jnp.int32),
            pltpu.VMEM((CH,), jnp.int32),
            pltpu.VMEM((CH,), jnp.int32),
            pltpu.VMEM((CH,), jnp.int32),
            pltpu.VMEM((CH,), jnp.int32),
            pltpu.SemaphoreType.DMA,
        ],
    )


def _ffn_body(te_ref, xs_ref, w1_ref, b1_ref, w2_ref, b2_ref, ys_ref):
    xb = xs_ref[...].astype(jnp.bfloat16)
    h = jnp.dot(xb, w1_ref[0], preferred_element_type=jnp.float32)
    h = h + b1_ref[0]
    h = 0.5 * h * (1.0 + lax.erf(h * 0.7071067811865476))
    y = jnp.dot(h.astype(jnp.bfloat16), w2_ref[0],
                preferred_element_type=jnp.float32)
    ys_ref[...] = y + b2_ref[0]


def _ffn(te, xs, w1b, b1, w2b, b2):
    grid_spec = pltpu.PrefetchScalarGridSpec(
        num_scalar_prefetch=1,
        grid=(NT,),
        in_specs=[
            pl.BlockSpec((T, DIM), lambda g, te_ref: (g, 0)),
            pl.BlockSpec((1, DIM, HID), lambda g, te_ref: (te_ref[g], 0, 0)),
            pl.BlockSpec((1, 1, HID), lambda g, te_ref: (te_ref[g], 0, 0)),
            pl.BlockSpec((1, HID, DIM), lambda g, te_ref: (te_ref[g], 0, 0)),
            pl.BlockSpec((1, 1, DIM), lambda g, te_ref: (te_ref[g], 0, 0)),
        ],
        out_specs=pl.BlockSpec((T, DIM), lambda g, te_ref: (g, 0)),
    )
    return pl.pallas_call(
        _ffn_body,
        grid_spec=grid_spec,
        out_shape=jax.ShapeDtypeStruct((M, DIM), jnp.float32),
        compiler_params=pltpu.CompilerParams(
            dimension_semantics=("arbitrary",)),
    )(te, xs, w1b, b1.reshape(E, 1, HID), w2b, b2.reshape(E, 1, DIM))


def _combine_body(ys_hbm, pos0_hbm, pos1_hbm, g0_hbm, g1_hbm, out_hbm,
             abuf, bbuf, obuf, p0v, p1v, g0v, g1v, sem):
    wid = lax.axis_index("s") * NC + lax.axis_index("c")
    t0 = wid * TOK_W
    lanes = lax.iota(jnp.int32, 16)

    def chunk(ci, carry):
        off = t0 + ci * CH2
        pltpu.sync_copy(pos0_hbm.at[pl.ds(off, CH2)], p0v)
        pltpu.sync_copy(pos1_hbm.at[pl.ds(off, CH2)], p1v)
        pltpu.sync_copy(g0_hbm.at[pl.ds(off, CH2)], g0v)
        pltpu.sync_copy(g1_hbm.at[pl.ds(off, CH2)], g1v)
        pltpu.async_copy(ys_hbm.at[p0v], abuf, sem).wait()
        pltpu.async_copy(ys_hbm.at[p1v], bbuf, sem).wait()
        g0reg = g0v[...]
        g1reg = g1v[...]

        def row(i, c2):
            iv = lanes * 0 + i
            s0 = _take16(g0reg, iv)
            s1 = _take16(g1reg, iv)

            def col(d, c3):
                cs = pl.ds(d * 16, 16)
                obuf[i, cs] = abuf[i, cs] * s0 + bbuf[i, cs] * s1
                return c3

            lax.fori_loop(0, DIM // 16, col, 0)
            return c2

        lax.fori_loop(0, CH2, row, 0)
        pltpu.sync_copy(obuf, out_hbm.at[pl.ds(off, CH2)])
        return carry

    lax.fori_loop(0, TOK_W // CH2, chunk, 0)


@functools.lru_cache(maxsize=None)
def _make_combine():
    mesh = plsc.VectorSubcoreMesh(core_axis_name="c", subcore_axis_name="s")
    return pl.kernel(
        _combine_body, mesh=mesh,
        out_type=jax.ShapeDtypeStruct((N, DIM), jnp.float32),
        scratch_types=[
            pltpu.VMEM((CH2, DIM), jnp.float32),
            pltpu.VMEM((CH2, DIM), jnp.float32),
            pltpu.VMEM((CH2, DIM), jnp.float32),
            pltpu.VMEM((CH2,), jnp.int32),
            pltpu.VMEM((CH2,), jnp.int32),
            pltpu.VMEM((CH2,), jnp.float32),
            pltpu.VMEM((CH2,), jnp.float32),
            pltpu.SemaphoreType.DMA,
        ],
    )


def kernel(x, Wr, W1, b1, W2, b2):
    x2 = x.reshape(N, DIM)
    e0, e1, r0, r1, g0, g1, counts16, aux = _router(x2, Wr)

    cnt = counts16.reshape(E16)[:E]
    pc = ((cnt + (T - 1)) // T) * T
    incl = jnp.cumsum(pc)
    base16 = jnp.concatenate(
        [incl - pc, jnp.zeros((E16 - E,), jnp.int32)])
    gstart = jnp.arange(NT, dtype=jnp.int32) * T
    te = jnp.minimum((gstart[:, None] >= incl[None, :]).sum(axis=1),
                     E - 1).astype(jnp.int32)

    xs, pos0, pos1 = _make_dispatch()(x2, e0.reshape(N), e1.reshape(N),
                                      r0.reshape(N), r1.reshape(N), base16)
    ys = _ffn(te, xs, W1.astype(jnp.bfloat16), b1,
              W2.astype(jnp.bfloat16), b2)
    out2 = _make_combine()(ys, pos0, pos1, g0.reshape(N), g1.reshape(N))
    return out2.reshape(B, S, DIM), aux.reshape(())


# speedup vs baseline: 5.5514x; 1.0219x over previous
import functools

import jax
import jax.numpy as jnp
from jax import lax
from jax.experimental import pallas as pl
from jax.experimental.pallas import tpu as pltpu
from jax.experimental.pallas import tpu_sc as plsc

B, S, DIM, HID, E, K = 2, 2048, 1024, 4096, 8, 2
N = B * S
A = N * K
T = 128
M = A + E * T
NT = M // T
TB = 512
NTR = N // TB
E16 = 16

NC, NS = 2, 16
NW = NC * NS
TOK_W = N // NW
CH = 32
CH2 = 16


def _router_body(x_ref, wr_ref, e0_ref, e1_ref, r0_ref, r1_ref,
                 g0_ref, g1_ref, cnt_ref, aux_ref, cnt_acc, psum_acc):
    i = pl.program_id(0)
    x = x_ref[...]
    logits = jnp.dot(x, wr_ref[...], preferred_element_type=jnp.float32)
    m = jnp.max(logits, axis=-1, keepdims=True)
    p = jnp.exp(logits - m)
    p = p / jnp.sum(p, axis=-1, keepdims=True)

    i0 = jnp.argmax(p, axis=-1)
    eidx = lax.broadcasted_iota(jnp.int32, (TB, E16), 1)
    oh0 = (eidx == i0[:, None]).astype(jnp.float32)
    pm = jnp.where(eidx[:, :E] == i0[:, None], -jnp.inf, p)
    i1 = jnp.argmax(pm, axis=-1)
    oh1 = (eidx == i1[:, None]).astype(jnp.float32)
    m0 = jnp.max(p, axis=-1)
    m1 = jnp.max(pm, axis=-1)
    den = m0 + m1 + 1e-9
    g0 = m0 / den
    g1 = m1 / den

    @pl.when(i == 0)
    def _():
        cnt_acc[...] = jnp.zeros_like(cnt_acc)
        psum_acc[...] = jnp.zeros_like(psum_acc)

    ohsum = oh0 + oh1
    rows = lax.broadcasted_iota(jnp.int32, (TB, TB), 0)
    cols = lax.broadcasted_iota(jnp.int32, (TB, TB), 1)
    ltri = (cols < rows).astype(jnp.float32)
    csum = jnp.dot(ltri, ohsum, preferred_element_type=jnp.float32)
    before = cnt_acc[...] + csum
    r0 = jnp.sum(before * oh0, axis=-1)
    r1 = jnp.sum(before * oh1, axis=-1)

    e0_ref[...] = i0.astype(jnp.int32)[:, None]
    e1_ref[...] = i1.astype(jnp.int32)[:, None]
    r0_ref[...] = r0.astype(jnp.int32)[:, None]
    r1_ref[...] = r1.astype(jnp.int32)[:, None]
    g0_ref[...] = g0[:, None]
    g1_ref[...] = g1[:, None]

    newcnt = cnt_acc[...] + jnp.sum(ohsum, axis=0, keepdims=True)
    cnt_acc[...] = newcnt
    newpsum = psum_acc[...] + jnp.sum(p, axis=0, keepdims=True)
    psum_acc[...] = newpsum

    @pl.when(i == NTR - 1)
    def _():
        cnt_ref[...] = newcnt.astype(jnp.int32)
        avg_tokens = newcnt[0, :E] / float(N * K)
        avg_prob = newpsum[0] / float(N)
        aux_ref[...] = (0.01 * E * jnp.sum(avg_tokens * avg_prob,
                                           keepdims=True))[:, None]


def _router(x2, Wr):
    iv = [jax.ShapeDtypeStruct((N, 1), jnp.int32)] * 4
    fv = [jax.ShapeDtypeStruct((N, 1), jnp.float32)] * 2
    return pl.pallas_call(
        _router_body,
        grid=(NTR,),
        in_specs=[
            pl.BlockSpec((TB, DIM), lambda i: (i, 0)),
            pl.BlockSpec((DIM, E), lambda i: (0, 0)),
        ],
        out_specs=[pl.BlockSpec((TB, 1), lambda i: (i, 0))] * 6
        + [pl.BlockSpec((1, E16), lambda i: (0, 0)),
           pl.BlockSpec((1, 1), lambda i: (0, 0))],
        out_shape=iv + fv + [jax.ShapeDtypeStruct((1, E16), jnp.int32),
                             jax.ShapeDtypeStruct((1, 1), jnp.float32)],
        scratch_shapes=[pltpu.VMEM((1, E16), jnp.float32),
                        pltpu.VMEM((1, E), jnp.float32)],
        compiler_params=pltpu.CompilerParams(
            dimension_semantics=("arbitrary",)),
    )(x2, Wr)


def _take16(vec, idx):
    return lax.gather(
        vec, idx[:, None],
        dimension_numbers=lax.GatherDimensionNumbers(
            offset_dims=(), collapsed_slice_dims=(0,), start_index_map=(0,)),
        slice_sizes=(1,),
        mode=lax.GatherScatterMode.PROMISE_IN_BOUNDS)

def _dispatch_body(x_hbm, e0_hbm, e1_hbm, r0_hbm, r1_hbm, base_hbm,
              xs_hbm, pos0_hbm, pos1_hbm,
              xbuf, basev, e0v, e1v, r0v, r1v, p0v, p1v, sem):
    wid = lax.axis_index("s") * NC + lax.axis_index("c")
    t0 = wid * TOK_W
    pltpu.sync_copy(base_hbm, basev)
    basereg = basev[...]

    def chunk(ci, carry):
        off = t0 + ci * CH
        pltpu.sync_copy(e0_hbm.at[pl.ds(off, CH)], e0v)
        pltpu.sync_copy(e1_hbm.at[pl.ds(off, CH)], e1v)
        pltpu.sync_copy(r0_hbm.at[pl.ds(off, CH)], r0v)
        pltpu.sync_copy(r1_hbm.at[pl.ds(off, CH)], r1v)
        for j in range(CH // 16):
            sl = pl.ds(j * 16, 16)
            p0v[sl] = _take16(basereg, e0v[sl]) + r0v[sl]
            p1v[sl] = _take16(basereg, e1v[sl]) + r1v[sl]
        pltpu.sync_copy(x_hbm.at[pl.ds(off, CH)], xbuf)
        c0 = pltpu.async_copy(xbuf, xs_hbm.at[p0v], sem)
        c1 = pltpu.async_copy(xbuf, xs_hbm.at[p1v], sem)
        pltpu.sync_copy(p0v, pos0_hbm.at[pl.ds(off, CH)])
        pltpu.sync_copy(p1v, pos1_hbm.at[pl.ds(off, CH)])
        c0.wait()
        c1.wait()
        return carry

    lax.fori_loop(0, TOK_W // CH, chunk, 0)


@functools.lru_cache(maxsize=None)
def _make_dispatch():
    mesh = plsc.VectorSubcoreMesh(core_axis_name="c", subcore_axis_name="s")
    return pl.kernel(
        _dispatch_body, mesh=mesh,
        out_type=[
            jax.ShapeDtypeStruct((M, DIM), jnp.float32),
            jax.ShapeDtypeStruct((N,), jnp.int32),
            jax.ShapeDtypeStruct((N,), jnp.int32),
        ],
        scratch_types=[
            pltpu.VMEM((CH, DIM), jnp.float32),
            pltpu.VMEM((E16,), jnp.int32),
            pltpu.VMEM((CH,), jnp.int32),
            pltpu.VMEM((CH,), jnp.int32),
            pltpu.VMEM((CH,), jnp.int32),
            pltpu.VMEM((CH,), jnp.int32),
            pltpu.VMEM((CH,), jnp.int32),
            pltpu.VMEM((CH,), jnp.int32),
            pltpu.SemaphoreType.DMA,
        ],
    )


CHID = 1024


def _ffn_body(te_ref, xs_ref, w1_ref, b1_ref, w2_ref, b2_ref, ys_ref):
    xb = xs_ref[...].astype(jnp.bfloat16)
    y = None
    for c in range(HID // CHID):
        sl = pl.ds(c * CHID, CHID)
        h = jnp.dot(xb, w1_ref[0, :, sl],
                    preferred_element_type=jnp.float32)
        h = h + b1_ref[0, :, sl]
        h = 0.5 * h * (1.0 + lax.erf(h * 0.7071067811865476))
        yc = jnp.dot(h, w2_ref[0, sl, :],
                     precision=lax.Precision.DEFAULT,
                     preferred_element_type=jnp.float32)
        y = yc if y is None else y + yc
    ys_ref[...] = y + b2_ref[0]


def _ffn(te, xs, w1b, b1, w2b, b2):
    grid_spec = pltpu.PrefetchScalarGridSpec(
        num_scalar_prefetch=1,
        grid=(NT,),
        in_specs=[
            pl.BlockSpec((T, DIM), lambda g, te_ref: (g, 0)),
            pl.BlockSpec((1, DIM, HID), lambda g, te_ref: (te_ref[g], 0, 0)),
            pl.BlockSpec((1, 1, HID), lambda g, te_ref: (te_ref[g], 0, 0)),
            pl.BlockSpec((1, HID, DIM), lambda g, te_ref: (te_ref[g], 0, 0)),
            pl.BlockSpec((1, 1, DIM), lambda g, te_ref: (te_ref[g], 0, 0)),
        ],
        out_specs=pl.BlockSpec((T, DIM), lambda g, te_ref: (g, 0)),
    )
    return pl.pallas_call(
        _ffn_body,
        grid_spec=grid_spec,
        out_shape=jax.ShapeDtypeStruct((M, DIM), jnp.float32),
        compiler_params=pltpu.CompilerParams(
            dimension_semantics=("arbitrary",),
            vmem_limit_bytes=120 * 1024 * 1024),
    )(te, xs, w1b, b1.reshape(E, 1, HID), w2b, b2.reshape(E, 1, DIM))


def _combine_body(ys_hbm, pos0_hbm, pos1_hbm, g0_hbm, g1_hbm, out_hbm,
             abuf, bbuf, obuf, p0v, p1v, g0v, g1v, sem):
    wid = lax.axis_index("s") * NC + lax.axis_index("c")
    t0 = wid * TOK_W
    lanes = lax.iota(jnp.int32, 16)

    def chunk(ci, carry):
        off = t0 + ci * CH2
        pltpu.sync_copy(pos0_hbm.at[pl.ds(off, CH2)], p0v)
        pltpu.sync_copy(pos1_hbm.at[pl.ds(off, CH2)], p1v)
        pltpu.sync_copy(g0_hbm.at[pl.ds(off, CH2)], g0v)
        pltpu.sync_copy(g1_hbm.at[pl.ds(off, CH2)], g1v)
        pltpu.async_copy(ys_hbm.at[p0v], abuf, sem).wait()
        pltpu.async_copy(ys_hbm.at[p1v], bbuf, sem).wait()
        g0reg = g0v[...]
        g1reg = g1v[...]

        def row(i, c2):
            iv = lanes * 0 + i
            s0 = _take16(g0reg, iv)
            s1 = _take16(g1reg, iv)

            def col(d, c3):
                for u in range(8):
                    cs = pl.ds((d * 8 + u) * 16, 16)
                    obuf[i, cs] = abuf[i, cs] * s0 + bbuf[i, cs] * s1
                return c3

            lax.fori_loop(0, DIM // (16 * 8), col, 0)
            return c2

        lax.fori_loop(0, CH2, row, 0)
        pltpu.sync_copy(obuf, out_hbm.at[pl.ds(off, CH2)])
        return carry

    lax.fori_loop(0, TOK_W // CH2, chunk, 0)


@functools.lru_cache(maxsize=None)
def _make_combine():
    mesh = plsc.VectorSubcoreMesh(core_axis_name="c", subcore_axis_name="s")
    return pl.kernel(
        _combine_body, mesh=mesh,
        out_type=jax.ShapeDtypeStruct((N, DIM), jnp.float32),
        scratch_types=[
            pltpu.VMEM((CH2, DIM), jnp.float32),
            pltpu.VMEM((CH2, DIM), jnp.float32),
            pltpu.VMEM((CH2, DIM), jnp.float32),
            pltpu.VMEM((CH2,), jnp.int32),
            pltpu.VMEM((CH2,), jnp.int32),
            pltpu.VMEM((CH2,), jnp.float32),
            pltpu.VMEM((CH2,), jnp.float32),
            pltpu.SemaphoreType.DMA,
        ],
    )


def kernel(x, Wr, W1, b1, W2, b2):
    x2 = x.reshape(N, DIM)
    e0, e1, r0, r1, g0, g1, counts16, aux = _router(x2, Wr)

    cnt = counts16.reshape(E16)[:E]
    pc = ((cnt + (T - 1)) // T) * T
    incl = jnp.cumsum(pc)
    base16 = jnp.concatenate(
        [incl - pc, jnp.zeros((E16 - E,), jnp.int32)])
    gstart = jnp.arange(NT, dtype=jnp.int32) * T
    te = jnp.minimum((gstart[:, None] >= incl[None, :]).sum(axis=1),
                     E - 1).astype(jnp.int32)

    xs, pos0, pos1 = _make_dispatch()(x2, e0.reshape(N), e1.reshape(N),
                                      r0.reshape(N), r1.reshape(N), base16)
    ys = _ffn(te, xs, W1.astype(jnp.bfloat16), b1, W2, b2)
    out2 = _make_combine()(ys, pos0, pos1, g0.reshape(N), g1.reshape(N))
    return out2.reshape(B, S, DIM), aux.reshape(())


# speedup vs baseline: 5.8413x; 1.0522x over previous
import functools

import jax
import jax.numpy as jnp
from jax import lax
from jax.experimental import pallas as pl
from jax.experimental.pallas import tpu as pltpu
from jax.experimental.pallas import tpu_sc as plsc

B, S, DIM, HID, E, K = 2, 2048, 1024, 4096, 8, 2
N = B * S
A = N * K
T = 128
M = A + E * T
NT = M // T
TB = 512
NTR = N // TB
E16 = 16

NC, NS = 2, 16
NW = NC * NS
TOK_W = N // NW
CH = 32
CH2 = 16


NTP = 128


def _router_body(x_ref, wr_ref, e0_ref, e1_ref, r0_ref, r1_ref,
                 g0_ref, g1_ref, base_ref, te_ref, aux_ref,
                 cnt_acc, psum_acc):
    i = pl.program_id(0)
    x = x_ref[...]
    logits = jnp.dot(x, wr_ref[...], preferred_element_type=jnp.float32)
    m = jnp.max(logits, axis=-1, keepdims=True)
    p = jnp.exp(logits - m)
    p = p / jnp.sum(p, axis=-1, keepdims=True)

    i0 = jnp.argmax(p, axis=-1)
    eidx = lax.broadcasted_iota(jnp.int32, (TB, E16), 1)
    oh0 = (eidx == i0[:, None]).astype(jnp.float32)
    pm = jnp.where(eidx[:, :E] == i0[:, None], -jnp.inf, p)
    i1 = jnp.argmax(pm, axis=-1)
    oh1 = (eidx == i1[:, None]).astype(jnp.float32)
    m0 = jnp.max(p, axis=-1)
    m1 = jnp.max(pm, axis=-1)
    den = m0 + m1 + 1e-9
    g0 = m0 / den
    g1 = m1 / den

    @pl.when(i == 0)
    def _():
        cnt_acc[...] = jnp.zeros_like(cnt_acc)
        psum_acc[...] = jnp.zeros_like(psum_acc)

    ohsum = oh0 + oh1
    rows = lax.broadcasted_iota(jnp.int32, (TB, TB), 0)
    cols = lax.broadcasted_iota(jnp.int32, (TB, TB), 1)
    ltri = (cols < rows).astype(jnp.float32)
    csum = jnp.dot(ltri, ohsum, preferred_element_type=jnp.float32)
    before = cnt_acc[...] + csum
    r0 = jnp.sum(before * oh0, axis=-1)
    r1 = jnp.sum(before * oh1, axis=-1)

    e0_ref[...] = i0.astype(jnp.int32)[:, None]
    e1_ref[...] = i1.astype(jnp.int32)[:, None]
    r0_ref[...] = r0.astype(jnp.int32)[:, None]
    r1_ref[...] = r1.astype(jnp.int32)[:, None]
    g0_ref[...] = g0[:, None]
    g1_ref[...] = g1[:, None]

    newcnt = cnt_acc[...] + jnp.sum(ohsum, axis=0, keepdims=True)
    cnt_acc[...] = newcnt
    newpsum = psum_acc[...] + jnp.sum(p, axis=0, keepdims=True)
    psum_acc[...] = newpsum

    @pl.when(i == NTR - 1)
    def _():
        avg_tokens = newcnt[0, :E] / float(N * K)
        avg_prob = newpsum[0] / float(N)
        aux_ref[...] = (0.01 * E * jnp.sum(avg_tokens * avg_prob,
                                           keepdims=True))[:, None]
        cnt_i = newcnt.astype(jnp.int32)
        pc = ((cnt_i + (T - 1)) // T) * T
        ut = (lax.broadcasted_iota(jnp.int32, (E16, E16), 0)
              <= lax.broadcasted_iota(jnp.int32, (E16, E16), 1))
        incl = jnp.dot(pc.astype(jnp.float32), ut.astype(jnp.float32),
                       preferred_element_type=jnp.float32).astype(jnp.int32)
        base_ref[...] = incl - pc
        gstart = lax.broadcasted_iota(jnp.int32, (NTP, E16), 0) * T
        has_e = lax.broadcasted_iota(jnp.int32, (NTP, E16), 1) < E
        ge = jnp.where(has_e, (gstart >= incl).astype(jnp.int32), 0)
        te_ref[...] = jnp.minimum(jnp.sum(ge, axis=1), E - 1)[:, None]


def _router(x2, Wr):
    iv = [jax.ShapeDtypeStruct((N, 1), jnp.int32)] * 4
    fv = [jax.ShapeDtypeStruct((N, 1), jnp.float32)] * 2
    return pl.pallas_call(
        _router_body,
        grid=(NTR,),
        in_specs=[
            pl.BlockSpec((TB, DIM), lambda i: (i, 0)),
            pl.BlockSpec((DIM, E), lambda i: (0, 0)),
        ],
        out_specs=[pl.BlockSpec((TB, 1), lambda i: (i, 0))] * 6
        + [pl.BlockSpec((1, E16), lambda i: (0, 0)),
           pl.BlockSpec((NTP, 1), lambda i: (0, 0)),
           pl.BlockSpec((1, 1), lambda i: (0, 0))],
        out_shape=iv + fv + [jax.ShapeDtypeStruct((1, E16), jnp.int32),
                             jax.ShapeDtypeStruct((NTP, 1), jnp.int32),
                             jax.ShapeDtypeStruct((1, 1), jnp.float32)],
        scratch_shapes=[pltpu.VMEM((1, E16), jnp.float32),
                        pltpu.VMEM((1, E), jnp.float32)],
        compiler_params=pltpu.CompilerParams(
            dimension_semantics=("arbitrary",)),
    )(x2, Wr)


def _take16(vec, idx):
    return lax.gather(
        vec, idx[:, None],
        dimension_numbers=lax.GatherDimensionNumbers(
            offset_dims=(), collapsed_slice_dims=(0,), start_index_map=(0,)),
        slice_sizes=(1,),
        mode=lax.GatherScatterMode.PROMISE_IN_BOUNDS)

def _dispatch_body(x_hbm, e0_hbm, e1_hbm, r0_hbm, r1_hbm, base_hbm,
              xs_hbm, pos0_hbm, pos1_hbm,
              xbuf, basev, e0v, e1v, r0v, r1v, p0v, p1v, sem):
    wid = lax.axis_index("s") * NC + lax.axis_index("c")
    t0 = wid * TOK_W
    pltpu.sync_copy(base_hbm, basev)
    basereg = basev[...]

    def chunk(ci, carry):
        off = t0 + ci * CH
        pltpu.sync_copy(e0_hbm.at[pl.ds(off, CH)], e0v)
        pltpu.sync_copy(e1_hbm.at[pl.ds(off, CH)], e1v)
        pltpu.sync_copy(r0_hbm.at[pl.ds(off, CH)], r0v)
        pltpu.sync_copy(r1_hbm.at[pl.ds(off, CH)], r1v)
        for j in range(CH // 16):
            sl = pl.ds(j * 16, 16)
            p0v[sl] = _take16(basereg, e0v[sl]) + r0v[sl]
            p1v[sl] = _take16(basereg, e1v[sl]) + r1v[sl]
        pltpu.sync_copy(x_hbm.at[pl.ds(off, CH)], xbuf)
        c0 = pltpu.async_copy(xbuf, xs_hbm.at[p0v], sem)
        c1 = pltpu.async_copy(xbuf, xs_hbm.at[p1v], sem)
        pltpu.sync_copy(p0v, pos0_hbm.at[pl.ds(off, CH)])
        pltpu.sync_copy(p1v, pos1_hbm.at[pl.ds(off, CH)])
        c0.wait()
        c1.wait()
        return carry

    lax.fori_loop(0, TOK_W // CH, chunk, 0)


@functools.lru_cache(maxsize=None)
def _make_dispatch():
    mesh = plsc.VectorSubcoreMesh(core_axis_name="c", subcore_axis_name="s")
    return pl.kernel(
        _dispatch_body, mesh=mesh,
        out_type=[
            jax.ShapeDtypeStruct((M, DIM), jnp.float32),
            jax.ShapeDtypeStruct((N,), jnp.int32),
            jax.ShapeDtypeStruct((N,), jnp.int32),
        ],
        scratch_types=[
            pltpu.VMEM((CH, DIM), jnp.float32),
            pltpu.VMEM((E16,), jnp.int32),
            pltpu.VMEM((CH,), jnp.int32),
            pltpu.VMEM((CH,), jnp.int32),
            pltpu.VMEM((CH,), jnp.int32),
            pltpu.VMEM((CH,), jnp.int32),
            pltpu.VMEM((CH,), jnp.int32),
            pltpu.VMEM((CH,), jnp.int32),
            pltpu.SemaphoreType.DMA,
        ],
    )


CHID = 1024


def _ffn_body(te_ref, xs_ref, w1_ref, b1_ref, w2_ref, b2_ref, ys_ref):
    xb = xs_ref[...].astype(jnp.bfloat16)
    y = None
    for c in range(HID // CHID):
        sl = pl.ds(c * CHID, CHID)
        h = jnp.dot(xb, w1_ref[0, :, sl],
                    preferred_element_type=jnp.float32)
        h = h + b1_ref[0, :, sl]
        h = 0.5 * h * (1.0 + lax.erf(h * 0.7071067811865476))
        yc = jnp.dot(h, w2_ref[0, sl, :],
                     precision=lax.Precision.DEFAULT,
                     preferred_element_type=jnp.float32)
        y = yc if y is None else y + yc
    ys_ref[...] = y + b2_ref[0]


def _ffn(te, xs, w1b, b1, w2b, b2):
    grid_spec = pltpu.PrefetchScalarGridSpec(
        num_scalar_prefetch=1,
        grid=(NT,),
        in_specs=[
            pl.BlockSpec((T, DIM), lambda g, te_ref: (g, 0)),
            pl.BlockSpec((1, DIM, HID), lambda g, te_ref: (te_ref[g], 0, 0)),
            pl.BlockSpec((1, 1, HID), lambda g, te_ref: (te_ref[g], 0, 0)),
            pl.BlockSpec((1, HID, DIM), lambda g, te_ref: (te_ref[g], 0, 0)),
            pl.BlockSpec((1, 1, DIM), lambda g, te_ref: (te_ref[g], 0, 0)),
        ],
        out_specs=pl.BlockSpec((T, DIM), lambda g, te_ref: (g, 0)),
    )
    return pl.pallas_call(
        _ffn_body,
        grid_spec=grid_spec,
        out_shape=jax.ShapeDtypeStruct((M, DIM), jnp.float32),
        compiler_params=pltpu.CompilerParams(
            dimension_semantics=("arbitrary",),
            vmem_limit_bytes=120 * 1024 * 1024),
    )(te, xs, w1b, b1.reshape(E, 1, HID), w2b, b2.reshape(E, 1, DIM))


def _combine_body(ys_hbm, pos0_hbm, pos1_hbm, g0_hbm, g1_hbm, out_hbm,
             abuf, bbuf, obuf, p0v, p1v, g0v, g1v, sem):
    wid = lax.axis_index("s") * NC + lax.axis_index("c")
    t0 = wid * TOK_W
    lanes = lax.iota(jnp.int32, 16)

    def chunk(ci, carry):
        off = t0 + ci * CH2
        pltpu.sync_copy(pos0_hbm.at[pl.ds(off, CH2)], p0v)
        pltpu.sync_copy(pos1_hbm.at[pl.ds(off, CH2)], p1v)
        pltpu.sync_copy(g0_hbm.at[pl.ds(off, CH2)], g0v)
        pltpu.sync_copy(g1_hbm.at[pl.ds(off, CH2)], g1v)
        pltpu.async_copy(ys_hbm.at[p0v], abuf, sem).wait()
        pltpu.async_copy(ys_hbm.at[p1v], bbuf, sem).wait()
        g0reg = g0v[...]
        g1reg = g1v[...]

        for i in range(CH2):
            iv = lanes * 0 + i
            s0 = _take16(g0reg, iv)
            s1 = _take16(g1reg, iv)

            def col(d, c3, i=i, s0=s0, s1=s1):
                for u in range(4):
                    cs = pl.ds((d * 4 + u) * 16, 16)
                    obuf[i, cs] = abuf[i, cs] * s0 + bbuf[i, cs] * s1
                return c3

            lax.fori_loop(0, DIM // 64, col, 0)
        pltpu.sync_copy(obuf, out_hbm.at[pl.ds(off, CH2)])
        return carry

    lax.fori_loop(0, TOK_W // CH2, chunk, 0)


@functools.lru_cache(maxsize=None)
def _make_combine():
    mesh = plsc.VectorSubcoreMesh(core_axis_name="c", subcore_axis_name="s")
    return pl.kernel(
        _combine_body, mesh=mesh,
        out_type=jax.ShapeDtypeStruct((N, DIM), jnp.float32),
        scratch_types=[
            pltpu.VMEM((CH2, DIM), jnp.float32),
            pltpu.VMEM((CH2, DIM), jnp.float32),
            pltpu.VMEM((CH2, DIM), jnp.float32),
            pltpu.VMEM((CH2,), jnp.int32),
            pltpu.VMEM((CH2,), jnp.int32),
            pltpu.VMEM((CH2,), jnp.float32),
            pltpu.VMEM((CH2,), jnp.float32),
            pltpu.SemaphoreType.DMA,
        ],
    )


def kernel(x, Wr, W1, b1, W2, b2):
    x2 = x.reshape(N, DIM)
    e0, e1, r0, r1, g0, g1, base16, te, aux = _router(x2, Wr)
    base16 = base16.reshape(E16)
    te = te.reshape(NTP)[:NT]

    xs, pos0, pos1 = _make_dispatch()(x2, e0.reshape(N), e1.reshape(N),
                                      r0.reshape(N), r1.reshape(N), base16)
    ys = _ffn(te, xs, W1.astype(jnp.bfloat16), b1, W2, b2)
    out2 = _make_combine()(ys, pos0, pos1, g0.reshape(N), g1.reshape(N))
    return out2.reshape(B, S, DIM), aux.reshape(())


# speedup vs baseline: 6.0447x; 1.0348x over previous
import functools

import jax
import jax.numpy as jnp
from jax import lax
from jax.experimental import pallas as pl
from jax.experimental.pallas import tpu as pltpu
from jax.experimental.pallas import tpu_sc as plsc

B, S, DIM, HID, E, K = 2, 2048, 1024, 4096, 8, 2
N = B * S
A = N * K
T = 128
M = A + E * T
NT = M // T
TB = 512
NTR = N // TB
E16 = 16

NC, NS = 2, 16
NW = NC * NS
TOK_W = N // NW
CH = 32
CH2 = 16


NTP = 128


def _router_body(x_ref, wr_ref, e0_ref, e1_ref, r0_ref, r1_ref,
                 g0_ref, g1_ref, base_ref, te_ref, aux_ref,
                 cnt_acc, psum_acc):
    i = pl.program_id(0)
    x = x_ref[...]
    logits = jnp.dot(x, wr_ref[...], preferred_element_type=jnp.float32)
    m = jnp.max(logits, axis=-1, keepdims=True)
    p = jnp.exp(logits - m)
    p = p / jnp.sum(p, axis=-1, keepdims=True)

    i0 = jnp.argmax(p, axis=-1)
    eidx = lax.broadcasted_iota(jnp.int32, (TB, E16), 1)
    oh0 = (eidx == i0[:, None]).astype(jnp.float32)
    pm = jnp.where(eidx[:, :E] == i0[:, None], -jnp.inf, p)
    i1 = jnp.argmax(pm, axis=-1)
    oh1 = (eidx == i1[:, None]).astype(jnp.float32)
    m0 = jnp.max(p, axis=-1)
    m1 = jnp.max(pm, axis=-1)
    den = m0 + m1 + 1e-9
    g0 = m0 / den
    g1 = m1 / den

    @pl.when(i == 0)
    def _():
        cnt_acc[...] = jnp.zeros_like(cnt_acc)
        psum_acc[...] = jnp.zeros_like(psum_acc)

    ohsum = oh0 + oh1
    rows = lax.broadcasted_iota(jnp.int32, (TB, TB), 0)
    cols = lax.broadcasted_iota(jnp.int32, (TB, TB), 1)
    ltri = (cols < rows).astype(jnp.float32)
    csum = jnp.dot(ltri, ohsum, preferred_element_type=jnp.float32)
    before = cnt_acc[...] + csum
    r0 = jnp.sum(before * oh0, axis=-1)
    r1 = jnp.sum(before * oh1, axis=-1)

    e0_ref[...] = i0.astype(jnp.int32)[:, None]
    e1_ref[...] = i1.astype(jnp.int32)[:, None]
    r0_ref[...] = r0.astype(jnp.int32)[:, None]
    r1_ref[...] = r1.astype(jnp.int32)[:, None]
    g0_ref[...] = g0[:, None]
    g1_ref[...] = g1[:, None]

    newcnt = cnt_acc[...] + jnp.sum(ohsum, axis=0, keepdims=True)
    cnt_acc[...] = newcnt
    newpsum = psum_acc[...] + jnp.sum(p, axis=0, keepdims=True)
    psum_acc[...] = newpsum

    @pl.when(i == NTR - 1)
    def _():
        avg_tokens = newcnt[0, :E] / float(N * K)
        avg_prob = newpsum[0] / float(N)
        aux_ref[...] = (0.01 * E * jnp.sum(avg_tokens * avg_prob,
                                           keepdims=True))[:, None]
        cnt_i = newcnt.astype(jnp.int32)
        pc = ((cnt_i + (T - 1)) // T) * T
        ut = (lax.broadcasted_iota(jnp.int32, (E16, E16), 0)
              <= lax.broadcasted_iota(jnp.int32, (E16, E16), 1))
        incl = jnp.dot(pc.astype(jnp.float32), ut.astype(jnp.float32),
                       preferred_element_type=jnp.float32).astype(jnp.int32)
        base_ref[...] = incl - pc
        gstart = lax.broadcasted_iota(jnp.int32, (NTP, E16), 0) * T
        has_e = lax.broadcasted_iota(jnp.int32, (NTP, E16), 1) < E
        ge = jnp.where(has_e, (gstart >= incl).astype(jnp.int32), 0)
        te_ref[...] = jnp.minimum(jnp.sum(ge, axis=1), E - 1)[:, None]


def _router(x2, Wr):
    iv = [jax.ShapeDtypeStruct((N, 1), jnp.int32)] * 4
    fv = [jax.ShapeDtypeStruct((N, 1), jnp.float32)] * 2
    return pl.pallas_call(
        _router_body,
        grid=(NTR,),
        in_specs=[
            pl.BlockSpec((TB, DIM), lambda i: (i, 0)),
            pl.BlockSpec((DIM, E), lambda i: (0, 0)),
        ],
        out_specs=[pl.BlockSpec((TB, 1), lambda i: (i, 0))] * 6
        + [pl.BlockSpec((1, E16), lambda i: (0, 0)),
           pl.BlockSpec((NTP, 1), lambda i: (0, 0)),
           pl.BlockSpec((1, 1), lambda i: (0, 0))],
        out_shape=iv + fv + [jax.ShapeDtypeStruct((1, E16), jnp.int32),
                             jax.ShapeDtypeStruct((NTP, 1), jnp.int32),
                             jax.ShapeDtypeStruct((1, 1), jnp.float32)],
        scratch_shapes=[pltpu.VMEM((1, E16), jnp.float32),
                        pltpu.VMEM((1, E), jnp.float32)],
        compiler_params=pltpu.CompilerParams(
            dimension_semantics=("arbitrary",)),
    )(x2, Wr)


def _take16(vec, idx):
    return lax.gather(
        vec, idx[:, None],
        dimension_numbers=lax.GatherDimensionNumbers(
            offset_dims=(), collapsed_slice_dims=(0,), start_index_map=(0,)),
        slice_sizes=(1,),
        mode=lax.GatherScatterMode.PROMISE_IN_BOUNDS)

NCHD = TOK_W // CH
NBUF = 3


def _dispatch_body(x_hbm, e0_hbm, e1_hbm, r0_hbm, r1_hbm, base_hbm,
                   xs_hbm, pos0_hbm, pos1_hbm,
                   xbuf, basev, e0v, e1v, r0v, r1v, p0v, p1v,
                   xsem0, xsem1, xsem2, ssem0, ssem1, ssem2):
    wid = lax.axis_index("s") * NC + lax.axis_index("c")
    t0 = wid * TOK_W
    pltpu.sync_copy(base_hbm, basev)
    basereg = basev[...]
    xsems = (xsem0, xsem1, xsem2)
    ssems = (ssem0, ssem1, ssem2)

    def prep(ci):
        off = t0 + ci * CH
        pltpu.sync_copy(e0_hbm.at[pl.ds(off, CH)], e0v)
        pltpu.sync_copy(e1_hbm.at[pl.ds(off, CH)], e1v)
        pltpu.sync_copy(r0_hbm.at[pl.ds(off, CH)], r0v)
        pltpu.sync_copy(r1_hbm.at[pl.ds(off, CH)], r1v)
        for j in range(CH // 16):
            sl = pl.ds(j * 16, 16)
            p0v[ci, sl] = _take16(basereg, e0v[sl]) + r0v[sl]
            p1v[ci, sl] = _take16(basereg, e1v[sl]) + r1v[sl]
        pltpu.sync_copy(p0v.at[ci], pos0_hbm.at[pl.ds(off, CH)])
        pltpu.sync_copy(p1v.at[ci], pos1_hbm.at[pl.ds(off, CH)])
        return pltpu.async_copy(x_hbm.at[pl.ds(off, CH)],
                                xbuf.at[ci % NBUF], xsems[ci % NBUF])

    loads = {0: prep(0), 1: prep(1)}
    scats = {}
    for ci in range(NCHD):
        b = ci % NBUF
        loads[ci].wait()
        scats[ci] = (
            pltpu.async_copy(xbuf.at[b], xs_hbm.at[p0v.at[ci]], ssems[b]),
            pltpu.async_copy(xbuf.at[b], xs_hbm.at[p1v.at[ci]], ssems[b]),
        )
        nxt = ci + 2
        if nxt < NCHD:
            if nxt - NBUF in scats:
                scats[nxt - NBUF][0].wait()
                scats[nxt - NBUF][1].wait()
                del scats[nxt - NBUF]
            loads[nxt] = prep(nxt)
    for pair in scats.values():
        pair[0].wait()
        pair[1].wait()


@functools.lru_cache(maxsize=None)
def _make_dispatch():
    mesh = plsc.VectorSubcoreMesh(core_axis_name="c", subcore_axis_name="s")
    return pl.kernel(
        _dispatch_body, mesh=mesh,
        out_type=[
            jax.ShapeDtypeStruct((M, DIM), jnp.float32),
            jax.ShapeDtypeStruct((N,), jnp.int32),
            jax.ShapeDtypeStruct((N,), jnp.int32),
        ],
        scratch_types=[
            pltpu.VMEM((NBUF, CH, DIM), jnp.float32),
            pltpu.VMEM((E16,), jnp.int32),
            pltpu.VMEM((CH,), jnp.int32),
            pltpu.VMEM((CH,), jnp.int32),
            pltpu.VMEM((CH,), jnp.int32),
            pltpu.VMEM((CH,), jnp.int32),
            pltpu.VMEM((NCHD, CH), jnp.int32),
            pltpu.VMEM((NCHD, CH), jnp.int32),
            pltpu.SemaphoreType.DMA,
            pltpu.SemaphoreType.DMA,
            pltpu.SemaphoreType.DMA,
            pltpu.SemaphoreType.DMA,
            pltpu.SemaphoreType.DMA,
            pltpu.SemaphoreType.DMA,
        ],
    )


CHID = 1024


def _ffn_body(te_ref, xs_ref, w1_ref, b1_ref, w2_ref, b2_ref, ys_ref):
    xb = xs_ref[...].astype(jnp.bfloat16)
    y = None
    for c in range(HID // CHID):
        sl = pl.ds(c * CHID, CHID)
        h = jnp.dot(xb, w1_ref[0, :, sl],
                    preferred_element_type=jnp.float32)
        h = h + b1_ref[0, :, sl]
        h = 0.5 * h * (1.0 + lax.erf(h * 0.7071067811865476))
        yc = jnp.dot(h, w2_ref[0, sl, :],
                     precision=lax.Precision.DEFAULT,
                     preferred_element_type=jnp.float32)
        y = yc if y is None else y + yc
    ys_ref[...] = y + b2_ref[0]


def _ffn(te, xs, w1b, b1, w2b, b2):
    grid_spec = pltpu.PrefetchScalarGridSpec(
        num_scalar_prefetch=1,
        grid=(NT,),
        in_specs=[
            pl.BlockSpec((T, DIM), lambda g, te_ref: (g, 0)),
            pl.BlockSpec((1, DIM, HID), lambda g, te_ref: (te_ref[g], 0, 0)),
            pl.BlockSpec((1, 1, HID), lambda g, te_ref: (te_ref[g], 0, 0)),
            pl.BlockSpec((1, HID, DIM), lambda g, te_ref: (te_ref[g], 0, 0)),
            pl.BlockSpec((1, 1, DIM), lambda g, te_ref: (te_ref[g], 0, 0)),
        ],
        out_specs=pl.BlockSpec((T, DIM), lambda g, te_ref: (g, 0)),
    )
    return pl.pallas_call(
        _ffn_body,
        grid_spec=grid_spec,
        out_shape=jax.ShapeDtypeStruct((M, DIM), jnp.float32),
        compiler_params=pltpu.CompilerParams(
            dimension_semantics=("arbitrary",),
            vmem_limit_bytes=120 * 1024 * 1024),
    )(te, xs, w1b, b1.reshape(E, 1, HID), w2b, b2.reshape(E, 1, DIM))


NCHC = TOK_W // CH2


def _combine_body(ys_hbm, pos0_hbm, pos1_hbm, g0_hbm, g1_hbm, out_hbm,
                  abuf, bbuf, obuf, p0v, p1v, g0v, g1v,
                  gsem0, gsem1, osem0, osem1):
    wid = lax.axis_index("s") * NC + lax.axis_index("c")
    t0 = wid * TOK_W
    lanes = lax.iota(jnp.int32, 16)
    gsems = (gsem0, gsem1)
    osems = (osem0, osem1)

    def gfire(ci):
        par = ci % 2
        off = t0 + ci * CH2
        pltpu.sync_copy(pos0_hbm.at[pl.ds(off, CH2)], p0v.at[par])
        pltpu.sync_copy(pos1_hbm.at[pl.ds(off, CH2)], p1v.at[par])
        pltpu.sync_copy(g0_hbm.at[pl.ds(off, CH2)], g0v.at[par])
        pltpu.sync_copy(g1_hbm.at[pl.ds(off, CH2)], g1v.at[par])
        return (pltpu.async_copy(ys_hbm.at[p0v.at[par]], abuf.at[par],
                                 gsems[par]),
                pltpu.async_copy(ys_hbm.at[p1v.at[par]], bbuf.at[par],
                                 gsems[par]))

    gats = {0: gfire(0)}
    osts = {}
    for ci in range(NCHC):
        par = ci % 2
        if ci + 1 < NCHC:
            gats[ci + 1] = gfire(ci + 1)
        gats[ci][0].wait()
        gats[ci][1].wait()
        if ci - 2 in osts:
            osts[ci - 2].wait()
            del osts[ci - 2]
        g0reg = g0v[par]
        g1reg = g1v[par]
        for i in range(CH2):
            iv = lanes * 0 + i
            s0 = _take16(g0reg, iv)
            s1 = _take16(g1reg, iv)

            def col(d, c3, par=par, i=i, s0=s0, s1=s1):
                for u in range(4):
                    cs = pl.ds((d * 4 + u) * 16, 16)
                    obuf[par, i, cs] = (abuf[par, i, cs] * s0
                                        + bbuf[par, i, cs] * s1)
                return c3

            lax.fori_loop(0, DIM // 64, col, 0)
        osts[ci] = pltpu.async_copy(obuf.at[par],
                                    out_hbm.at[pl.ds(t0 + ci * CH2, CH2)],
                                    osems[par])
    for o in osts.values():
        o.wait()


@functools.lru_cache(maxsize=None)
def _make_combine():
    mesh = plsc.VectorSubcoreMesh(core_axis_name="c", subcore_axis_name="s")
    return pl.kernel(
        _combine_body, mesh=mesh,
        out_type=jax.ShapeDtypeStruct((N, DIM), jnp.float32),
        scratch_types=[
            pltpu.VMEM((2, CH2, DIM), jnp.float32),
            pltpu.VMEM((2, CH2, DIM), jnp.float32),
            pltpu.VMEM((2, CH2, DIM), jnp.float32),
            pltpu.VMEM((2, CH2), jnp.int32),
            pltpu.VMEM((2, CH2), jnp.int32),
            pltpu.VMEM((2, CH2), jnp.float32),
            pltpu.VMEM((2, CH2), jnp.float32),
            pltpu.SemaphoreType.DMA,
            pltpu.SemaphoreType.DMA,
            pltpu.SemaphoreType.DMA,
            pltpu.SemaphoreType.DMA,
        ],
    )


def kernel(x, Wr, W1, b1, W2, b2):
    x2 = x.reshape(N, DIM)
    e0, e1, r0, r1, g0, g1, base16, te, aux = _router(x2, Wr)
    base16 = base16.reshape(E16)
    te = te.reshape(NTP)[:NT]

    xs, pos0, pos1 = _make_dispatch()(x2, e0.reshape(N), e1.reshape(N),
                                      r0.reshape(N), r1.reshape(N), base16)
    ys = _ffn(te, xs, W1.astype(jnp.bfloat16), b1, W2, b2)
    out2 = _make_combine()(ys, pos0, pos1, g0.reshape(N), g1.reshape(N))
    return out2.reshape(B, S, DIM), aux.reshape(())


# speedup vs baseline: 6.3117x; 1.0442x over previous
import functools

import jax
import jax.numpy as jnp
from jax import lax
from jax.experimental import pallas as pl
from jax.experimental.pallas import tpu as pltpu
from jax.experimental.pallas import tpu_sc as plsc

B, S, DIM, HID, E, K = 2, 2048, 1024, 4096, 8, 2
N = B * S
A = N * K
T = 256
M = A + E * T
NT = M // T
TB = 512
NTR = N // TB
E16 = 16

NC, NS = 2, 16
NW = NC * NS
TOK_W = N // NW
CH = 32
CH2 = 16


NTP = 128


def _router_body(x_ref, wr_ref, e0_ref, e1_ref, r0_ref, r1_ref,
                 g0_ref, g1_ref, base_ref, te_ref, aux_ref,
                 cnt_acc, psum_acc):
    i = pl.program_id(0)
    x = x_ref[...]
    logits = jnp.dot(x, wr_ref[...], preferred_element_type=jnp.float32)
    m = jnp.max(logits, axis=-1, keepdims=True)
    p = jnp.exp(logits - m)
    p = p / jnp.sum(p, axis=-1, keepdims=True)

    i0 = jnp.argmax(p, axis=-1)
    eidx = lax.broadcasted_iota(jnp.int32, (TB, E16), 1)
    oh0 = (eidx == i0[:, None]).astype(jnp.float32)
    pm = jnp.where(eidx[:, :E] == i0[:, None], -jnp.inf, p)
    i1 = jnp.argmax(pm, axis=-1)
    oh1 = (eidx == i1[:, None]).astype(jnp.float32)
    m0 = jnp.max(p, axis=-1)
    m1 = jnp.max(pm, axis=-1)
    den = m0 + m1 + 1e-9
    g0 = m0 / den
    g1 = m1 / den

    @pl.when(i == 0)
    def _():
        cnt_acc[...] = jnp.zeros_like(cnt_acc)
        psum_acc[...] = jnp.zeros_like(psum_acc)

    ohsum = oh0 + oh1
    rows = lax.broadcasted_iota(jnp.int32, (TB, TB), 0)
    cols = lax.broadcasted_iota(jnp.int32, (TB, TB), 1)
    ltri = (cols < rows).astype(jnp.float32)
    csum = jnp.dot(ltri, ohsum, preferred_element_type=jnp.float32)
    before = cnt_acc[...] + csum
    r0 = jnp.sum(before * oh0, axis=-1)
    r1 = jnp.sum(before * oh1, axis=-1)

    e0_ref[...] = i0.astype(jnp.int32)[:, None]
    e1_ref[...] = i1.astype(jnp.int32)[:, None]
    r0_ref[...] = r0.astype(jnp.int32)[:, None]
    r1_ref[...] = r1.astype(jnp.int32)[:, None]
    g0_ref[...] = g0[:, None]
    g1_ref[...] = g1[:, None]

    newcnt = cnt_acc[...] + jnp.sum(ohsum, axis=0, keepdims=True)
    cnt_acc[...] = newcnt
    newpsum = psum_acc[...] + jnp.sum(p, axis=0, keepdims=True)
    psum_acc[...] = newpsum

    @pl.when(i == NTR - 1)
    def _():
        avg_tokens = newcnt[0, :E] / float(N * K)
        avg_prob = newpsum[0] / float(N)
        aux_ref[...] = (0.01 * E * jnp.sum(avg_tokens * avg_prob,
                                           keepdims=True))[:, None]
        cnt_i = newcnt.astype(jnp.int32)
        pc = ((cnt_i + (T - 1)) // T) * T
        ut = (lax.broadcasted_iota(jnp.int32, (E16, E16), 0)
              <= lax.broadcasted_iota(jnp.int32, (E16, E16), 1))
        incl = jnp.dot(pc.astype(jnp.float32), ut.astype(jnp.float32),
                       preferred_element_type=jnp.float32).astype(jnp.int32)
        base_ref[...] = incl - pc
        gstart = lax.broadcasted_iota(jnp.int32, (NTP, E16), 0) * T
        has_e = lax.broadcasted_iota(jnp.int32, (NTP, E16), 1) < E
        ge = jnp.where(has_e, (gstart >= incl).astype(jnp.int32), 0)
        te_ref[...] = jnp.minimum(jnp.sum(ge, axis=1), E - 1)[:, None]


def _router(x2, Wr):
    iv = [jax.ShapeDtypeStruct((N, 1), jnp.int32)] * 4
    fv = [jax.ShapeDtypeStruct((N, 1), jnp.float32)] * 2
    return pl.pallas_call(
        _router_body,
        grid=(NTR,),
        in_specs=[
            pl.BlockSpec((TB, DIM), lambda i: (i, 0)),
            pl.BlockSpec((DIM, E), lambda i: (0, 0)),
        ],
        out_specs=[pl.BlockSpec((TB, 1), lambda i: (i, 0))] * 6
        + [pl.BlockSpec((1, E16), lambda i: (0, 0)),
           pl.BlockSpec((NTP, 1), lambda i: (0, 0)),
           pl.BlockSpec((1, 1), lambda i: (0, 0))],
        out_shape=iv + fv + [jax.ShapeDtypeStruct((1, E16), jnp.int32),
                             jax.ShapeDtypeStruct((NTP, 1), jnp.int32),
                             jax.ShapeDtypeStruct((1, 1), jnp.float32)],
        scratch_shapes=[pltpu.VMEM((1, E16), jnp.float32),
                        pltpu.VMEM((1, E), jnp.float32)],
        compiler_params=pltpu.CompilerParams(
            dimension_semantics=("arbitrary",)),
    )(x2, Wr)


def _take16(vec, idx):
    return lax.gather(
        vec, idx[:, None],
        dimension_numbers=lax.GatherDimensionNumbers(
            offset_dims=(), collapsed_slice_dims=(0,), start_index_map=(0,)),
        slice_sizes=(1,),
        mode=lax.GatherScatterMode.PROMISE_IN_BOUNDS)

NCHD = TOK_W // CH
NBUF = 3


def _dispatch_body(x_hbm, e0_hbm, e1_hbm, r0_hbm, r1_hbm, base_hbm,
                   xs_hbm, pos0_hbm, pos1_hbm,
                   xbuf, basev, e0v, e1v, r0v, r1v, p0v, p1v,
                   xsem0, xsem1, xsem2, ssem0, ssem1, ssem2):
    wid = lax.axis_index("s") * NC + lax.axis_index("c")
    t0 = wid * TOK_W
    pltpu.sync_copy(base_hbm, basev)
    basereg = basev[...]
    xsems = (xsem0, xsem1, xsem2)
    ssems = (ssem0, ssem1, ssem2)

    def prep(ci):
        off = t0 + ci * CH
        pltpu.sync_copy(e0_hbm.at[pl.ds(off, CH)], e0v)
        pltpu.sync_copy(e1_hbm.at[pl.ds(off, CH)], e1v)
        pltpu.sync_copy(r0_hbm.at[pl.ds(off, CH)], r0v)
        pltpu.sync_copy(r1_hbm.at[pl.ds(off, CH)], r1v)
        for j in range(CH // 16):
            sl = pl.ds(j * 16, 16)
            p0v[ci, sl] = _take16(basereg, e0v[sl]) + r0v[sl]
            p1v[ci, sl] = _take16(basereg, e1v[sl]) + r1v[sl]
        pltpu.sync_copy(p0v.at[ci], pos0_hbm.at[pl.ds(off, CH)])
        pltpu.sync_copy(p1v.at[ci], pos1_hbm.at[pl.ds(off, CH)])
        return pltpu.async_copy(x_hbm.at[pl.ds(off, CH)],
                                xbuf.at[ci % NBUF], xsems[ci % NBUF])

    loads = {0: prep(0), 1: prep(1)}
    scats = {}
    for ci in range(NCHD):
        b = ci % NBUF
        loads[ci].wait()
        scats[ci] = (
            pltpu.async_copy(xbuf.at[b], xs_hbm.at[p0v.at[ci]], ssems[b]),
            pltpu.async_copy(xbuf.at[b], xs_hbm.at[p1v.at[ci]], ssems[b]),
        )
        nxt = ci + 2
        if nxt < NCHD:
            if nxt - NBUF in scats:
                scats[nxt - NBUF][0].wait()
                scats[nxt - NBUF][1].wait()
                del scats[nxt - NBUF]
            loads[nxt] = prep(nxt)
    for pair in scats.values():
        pair[0].wait()
        pair[1].wait()


@functools.lru_cache(maxsize=None)
def _make_dispatch():
    mesh = plsc.VectorSubcoreMesh(core_axis_name="c", subcore_axis_name="s")
    return pl.kernel(
        _dispatch_body, mesh=mesh,
        out_type=[
            jax.ShapeDtypeStruct((M, DIM), jnp.float32),
            jax.ShapeDtypeStruct((N,), jnp.int32),
            jax.ShapeDtypeStruct((N,), jnp.int32),
        ],
        scratch_types=[
            pltpu.VMEM((NBUF, CH, DIM), jnp.float32),
            pltpu.VMEM((E16,), jnp.int32),
            pltpu.VMEM((CH,), jnp.int32),
            pltpu.VMEM((CH,), jnp.int32),
            pltpu.VMEM((CH,), jnp.int32),
            pltpu.VMEM((CH,), jnp.int32),
            pltpu.VMEM((NCHD, CH), jnp.int32),
            pltpu.VMEM((NCHD, CH), jnp.int32),
            pltpu.SemaphoreType.DMA,
            pltpu.SemaphoreType.DMA,
            pltpu.SemaphoreType.DMA,
            pltpu.SemaphoreType.DMA,
            pltpu.SemaphoreType.DMA,
            pltpu.SemaphoreType.DMA,
        ],
    )


CHID = 1024


def _ffn_body(te_ref, xs_ref, w1_ref, b1_ref, w2_ref, b2_ref, ys_ref):
    xb = xs_ref[...].astype(jnp.bfloat16)
    y = None
    for c in range(HID // CHID):
        sl = pl.ds(c * CHID, CHID)
        h = jnp.dot(xb, w1_ref[0, :, sl],
                    preferred_element_type=jnp.float32)
        h = h + b1_ref[0, :, sl]
        h = 0.5 * h * (1.0 + lax.erf(h * 0.7071067811865476))
        yc = jnp.dot(h, w2_ref[0, sl, :],
                     precision=lax.Precision.DEFAULT,
                     preferred_element_type=jnp.float32)
        y = yc if y is None else y + yc
    ys_ref[...] = y + b2_ref[0]


def _ffn(te, xs, w1b, b1, w2b, b2):
    grid_spec = pltpu.PrefetchScalarGridSpec(
        num_scalar_prefetch=1,
        grid=(NT,),
        in_specs=[
            pl.BlockSpec((T, DIM), lambda g, te_ref: (g, 0)),
            pl.BlockSpec((1, DIM, HID), lambda g, te_ref: (te_ref[g], 0, 0)),
            pl.BlockSpec((1, 1, HID), lambda g, te_ref: (te_ref[g], 0, 0)),
            pl.BlockSpec((1, HID, DIM), lambda g, te_ref: (te_ref[g], 0, 0)),
            pl.BlockSpec((1, 1, DIM), lambda g, te_ref: (te_ref[g], 0, 0)),
        ],
        out_specs=pl.BlockSpec((T, DIM), lambda g, te_ref: (g, 0)),
    )
    return pl.pallas_call(
        _ffn_body,
        grid_spec=grid_spec,
        out_shape=jax.ShapeDtypeStruct((M, DIM), jnp.float32),
        compiler_params=pltpu.CompilerParams(
            dimension_semantics=("arbitrary",),
            vmem_limit_bytes=120 * 1024 * 1024),
    )(te, xs, w1b, b1.reshape(E, 1, HID), w2b, b2.reshape(E, 1, DIM))


NCHC = TOK_W // CH2


def _combine_body(ys_hbm, pos0_hbm, pos1_hbm, g0_hbm, g1_hbm, out_hbm,
                  abuf, bbuf, obuf, p0v, p1v, g0v, g1v,
                  gsem0, gsem1, osem0, osem1):
    wid = lax.axis_index("s") * NC + lax.axis_index("c")
    t0 = wid * TOK_W
    lanes = lax.iota(jnp.int32, 16)
    gsems = (gsem0, gsem1)
    osems = (osem0, osem1)

    def gfire(ci):
        par = ci % 2
        off = t0 + ci * CH2
        pltpu.sync_copy(pos0_hbm.at[pl.ds(off, CH2)], p0v.at[par])
        pltpu.sync_copy(pos1_hbm.at[pl.ds(off, CH2)], p1v.at[par])
        pltpu.sync_copy(g0_hbm.at[pl.ds(off, CH2)], g0v.at[par])
        pltpu.sync_copy(g1_hbm.at[pl.ds(off, CH2)], g1v.at[par])
        return (pltpu.async_copy(ys_hbm.at[p0v.at[par]], abuf.at[par],
                                 gsems[par]),
                pltpu.async_copy(ys_hbm.at[p1v.at[par]], bbuf.at[par],
                                 gsems[par]))

    gats = {0: gfire(0)}
    osts = {}
    for ci in range(NCHC):
        par = ci % 2
        if ci + 1 < NCHC:
            gats[ci + 1] = gfire(ci + 1)
        gats[ci][0].wait()
        gats[ci][1].wait()
        if ci - 2 in osts:
            osts[ci - 2].wait()
            del osts[ci - 2]
        g0reg = g0v[par]
        g1reg = g1v[par]
        for i in range(CH2):
            iv = lanes * 0 + i
            s0 = _take16(g0reg, iv)
            s1 = _take16(g1reg, iv)

            def col(d, c3, par=par, i=i, s0=s0, s1=s1):
                for u in range(4):
                    cs = pl.ds((d * 4 + u) * 16, 16)
                    obuf[par, i, cs] = (abuf[par, i, cs] * s0
                                        + bbuf[par, i, cs] * s1)
                return c3

            lax.fori_loop(0, DIM // 64, col, 0)
        osts[ci] = pltpu.async_copy(obuf.at[par],
                                    out_hbm.at[pl.ds(t0 + ci * CH2, CH2)],
                                    osems[par])
    for o in osts.values():
        o.wait()


@functools.lru_cache(maxsize=None)
def _make_combine():
    mesh = plsc.VectorSubcoreMesh(core_axis_name="c", subcore_axis_name="s")
    return pl.kernel(
        _combine_body, mesh=mesh,
        out_type=jax.ShapeDtypeStruct((N, DIM), jnp.float32),
        scratch_types=[
            pltpu.VMEM((2, CH2, DIM), jnp.float32),
            pltpu.VMEM((2, CH2, DIM), jnp.float32),
            pltpu.VMEM((2, CH2, DIM), jnp.float32),
            pltpu.VMEM((2, CH2), jnp.int32),
            pltpu.VMEM((2, CH2), jnp.int32),
            pltpu.VMEM((2, CH2), jnp.float32),
            pltpu.VMEM((2, CH2), jnp.float32),
            pltpu.SemaphoreType.DMA,
            pltpu.SemaphoreType.DMA,
            pltpu.SemaphoreType.DMA,
            pltpu.SemaphoreType.DMA,
        ],
    )


def kernel(x, Wr, W1, b1, W2, b2):
    x2 = x.reshape(N, DIM)
    e0, e1, r0, r1, g0, g1, base16, te, aux = _router(x2, Wr)
    base16 = base16.reshape(E16)
    te = te.reshape(NTP)[:NT]

    xs, pos0, pos1 = _make_dispatch()(x2, e0.reshape(N), e1.reshape(N),
                                      r0.reshape(N), r1.reshape(N), base16)
    ys = _ffn(te, xs, W1.astype(jnp.bfloat16), b1, W2, b2)
    out2 = _make_combine()(ys, pos0, pos1, g0.reshape(N), g1.reshape(N))
    return out2.reshape(B, S, DIM), aux.reshape(())


# speedup vs baseline: 6.5966x; 1.0451x over previous
import functools

import jax
import jax.numpy as jnp
from jax import lax
from jax.experimental import pallas as pl
from jax.experimental.pallas import tpu as pltpu
from jax.experimental.pallas import tpu_sc as plsc

B, S, DIM, HID, E, K = 2, 2048, 1024, 4096, 8, 2
N = B * S
A = N * K
T = 256
M = A + E * T
NT = M // T
TB = 512
NTR = N // TB
E16 = 16

NC, NS = 2, 16
NW = NC * NS
TOK_W = N // NW
CH = 32
CH2 = 16


NTP = 128


def _router_body(x_ref, wr_ref, e0_ref, e1_ref, r0_ref, r1_ref,
                 g0_ref, g1_ref, base_ref, te_ref, aux_ref,
                 cnt_acc, psum_acc):
    i = pl.program_id(0)
    x = x_ref[...]
    logits = jnp.dot(x, wr_ref[...], preferred_element_type=jnp.float32)
    m = jnp.max(logits, axis=-1, keepdims=True)
    p = jnp.exp(logits - m)
    p = p / jnp.sum(p, axis=-1, keepdims=True)

    i0 = jnp.argmax(p, axis=-1)
    eidx = lax.broadcasted_iota(jnp.int32, (TB, E16), 1)
    oh0 = (eidx == i0[:, None]).astype(jnp.float32)
    pm = jnp.where(eidx[:, :E] == i0[:, None], -jnp.inf, p)
    i1 = jnp.argmax(pm, axis=-1)
    oh1 = (eidx == i1[:, None]).astype(jnp.float32)
    m0 = jnp.max(p, axis=-1)
    m1 = jnp.max(pm, axis=-1)
    den = m0 + m1 + 1e-9
    g0 = m0 / den
    g1 = m1 / den

    @pl.when(i == 0)
    def _():
        cnt_acc[...] = jnp.zeros_like(cnt_acc)
        psum_acc[...] = jnp.zeros_like(psum_acc)

    ohsum = oh0 + oh1
    rows = lax.broadcasted_iota(jnp.int32, (TB, TB), 0)
    cols = lax.broadcasted_iota(jnp.int32, (TB, TB), 1)
    ltri = (cols < rows).astype(jnp.float32)
    csum = jnp.dot(ltri, ohsum, preferred_element_type=jnp.float32)
    before = cnt_acc[...] + csum
    r0 = jnp.sum(before * oh0, axis=-1)
    r1 = jnp.sum(before * oh1, axis=-1)

    e0_ref[...] = i0.astype(jnp.int32)[:, None]
    e1_ref[...] = i1.astype(jnp.int32)[:, None]
    r0_ref[...] = r0.astype(jnp.int32)[:, None]
    r1_ref[...] = r1.astype(jnp.int32)[:, None]
    g0_ref[...] = g0[:, None]
    g1_ref[...] = g1[:, None]

    newcnt = cnt_acc[...] + jnp.sum(ohsum, axis=0, keepdims=True)
    cnt_acc[...] = newcnt
    newpsum = psum_acc[...] + jnp.sum(p, axis=0, keepdims=True)
    psum_acc[...] = newpsum

    @pl.when(i == NTR - 1)
    def _():
        avg_tokens = newcnt[0, :E] / float(N * K)
        avg_prob = newpsum[0] / float(N)
        aux_ref[...] = (0.01 * E * jnp.sum(avg_tokens * avg_prob,
                                           keepdims=True))[:, None]
        cnt_i = newcnt.astype(jnp.int32)
        pc = ((cnt_i + (T - 1)) // T) * T
        ut = (lax.broadcasted_iota(jnp.int32, (E16, E16), 0)
              <= lax.broadcasted_iota(jnp.int32, (E16, E16), 1))
        incl = jnp.dot(pc.astype(jnp.float32), ut.astype(jnp.float32),
                       preferred_element_type=jnp.float32).astype(jnp.int32)
        base_ref[...] = incl - pc
        gstart = lax.broadcasted_iota(jnp.int32, (NTP, E16), 0) * T
        eidx2 = lax.broadcasted_iota(jnp.int32, (NTP, E16), 1)
        ge = jnp.where(eidx2 < E, (gstart >= incl).astype(jnp.int32), 0)
        tev = jnp.minimum(jnp.sum(ge, axis=1), E - 1)
        totals = jnp.sum(jnp.where(eidx2 == E - 1, incl, 0), axis=1)
        te_ref[...] = jnp.where(gstart[:, 0] >= totals, E, tev)[:, None]


def _router(x2, Wr):
    iv = [jax.ShapeDtypeStruct((N, 1), jnp.int32)] * 4
    fv = [jax.ShapeDtypeStruct((N, 1), jnp.float32)] * 2
    return pl.pallas_call(
        _router_body,
        grid=(NTR,),
        in_specs=[
            pl.BlockSpec((TB, DIM), lambda i: (i, 0)),
            pl.BlockSpec((DIM, E), lambda i: (0, 0)),
        ],
        out_specs=[pl.BlockSpec((TB, 1), lambda i: (i, 0))] * 6
        + [pl.BlockSpec((1, E16), lambda i: (0, 0)),
           pl.BlockSpec((NTP, 1), lambda i: (0, 0)),
           pl.BlockSpec((1, 1), lambda i: (0, 0))],
        out_shape=iv + fv + [jax.ShapeDtypeStruct((1, E16), jnp.int32),
                             jax.ShapeDtypeStruct((NTP, 1), jnp.int32),
                             jax.ShapeDtypeStruct((1, 1), jnp.float32)],
        scratch_shapes=[pltpu.VMEM((1, E16), jnp.float32),
                        pltpu.VMEM((1, E), jnp.float32)],
        compiler_params=pltpu.CompilerParams(
            dimension_semantics=("arbitrary",)),
    )(x2, Wr)


def _take16(vec, idx):
    return lax.gather(
        vec, idx[:, None],
        dimension_numbers=lax.GatherDimensionNumbers(
            offset_dims=(), collapsed_slice_dims=(0,), start_index_map=(0,)),
        slice_sizes=(1,),
        mode=lax.GatherScatterMode.PROMISE_IN_BOUNDS)

NCHD = TOK_W // CH
NBUF = 3


def _dispatch_body(x_hbm, e0_hbm, e1_hbm, r0_hbm, r1_hbm, base_hbm,
                   xs_hbm, pos0_hbm, pos1_hbm,
                   xbuf, basev, e0v, e1v, r0v, r1v, p0v, p1v,
                   xsem0, xsem1, xsem2, ssem0, ssem1, ssem2):
    wid = lax.axis_index("s") * NC + lax.axis_index("c")
    t0 = wid * TOK_W
    pltpu.sync_copy(base_hbm, basev)
    basereg = basev[...]
    xsems = (xsem0, xsem1, xsem2)
    ssems = (ssem0, ssem1, ssem2)

    def prep(ci):
        off = t0 + ci * CH
        pltpu.sync_copy(e0_hbm.at[pl.ds(off, CH)], e0v)
        pltpu.sync_copy(e1_hbm.at[pl.ds(off, CH)], e1v)
        pltpu.sync_copy(r0_hbm.at[pl.ds(off, CH)], r0v)
        pltpu.sync_copy(r1_hbm.at[pl.ds(off, CH)], r1v)
        for j in range(CH // 16):
            sl = pl.ds(j * 16, 16)
            p0v[ci, sl] = _take16(basereg, e0v[sl]) + r0v[sl]
            p1v[ci, sl] = _take16(basereg, e1v[sl]) + r1v[sl]
        pltpu.sync_copy(p0v.at[ci], pos0_hbm.at[pl.ds(off, CH)])
        pltpu.sync_copy(p1v.at[ci], pos1_hbm.at[pl.ds(off, CH)])
        return pltpu.async_copy(x_hbm.at[pl.ds(off, CH)],
                                xbuf.at[ci % NBUF], xsems[ci % NBUF])

    loads = {0: prep(0), 1: prep(1)}
    scats = {}
    for ci in range(NCHD):
        b = ci % NBUF
        loads[ci].wait()
        scats[ci] = (
            pltpu.async_copy(xbuf.at[b], xs_hbm.at[p0v.at[ci]], ssems[b]),
            pltpu.async_copy(xbuf.at[b], xs_hbm.at[p1v.at[ci]], ssems[b]),
        )
        nxt = ci + 2
        if nxt < NCHD:
            if nxt - NBUF in scats:
                scats[nxt - NBUF][0].wait()
                scats[nxt - NBUF][1].wait()
                del scats[nxt - NBUF]
            loads[nxt] = prep(nxt)
    for pair in scats.values():
        pair[0].wait()
        pair[1].wait()


@functools.lru_cache(maxsize=None)
def _make_dispatch():
    mesh = plsc.VectorSubcoreMesh(core_axis_name="c", subcore_axis_name="s")
    return pl.kernel(
        _dispatch_body, mesh=mesh,
        out_type=[
            jax.ShapeDtypeStruct((M, DIM), jnp.float32),
            jax.ShapeDtypeStruct((N,), jnp.int32),
            jax.ShapeDtypeStruct((N,), jnp.int32),
        ],
        scratch_types=[
            pltpu.VMEM((NBUF, CH, DIM), jnp.float32),
            pltpu.VMEM((E16,), jnp.int32),
            pltpu.VMEM((CH,), jnp.int32),
            pltpu.VMEM((CH,), jnp.int32),
            pltpu.VMEM((CH,), jnp.int32),
            pltpu.VMEM((CH,), jnp.int32),
            pltpu.VMEM((NCHD, CH), jnp.int32),
            pltpu.VMEM((NCHD, CH), jnp.int32),
            pltpu.SemaphoreType.DMA,
            pltpu.SemaphoreType.DMA,
            pltpu.SemaphoreType.DMA,
            pltpu.SemaphoreType.DMA,
            pltpu.SemaphoreType.DMA,
            pltpu.SemaphoreType.DMA,
        ],
    )


CHID = 1024


def _ffn_body(te_ref, xs_ref, w1_ref, b1_ref, w2_ref, b2_ref, ys_ref):
    g = pl.program_id(0)

    @pl.when(te_ref[g] < E)
    def _():
        xb = xs_ref[...].astype(jnp.bfloat16)
        y = None
        for c in range(HID // CHID):
            sl = pl.ds(c * CHID, CHID)
            h = jnp.dot(xb, w1_ref[0, :, sl],
                        preferred_element_type=jnp.float32)
            h = h + b1_ref[0, :, sl]
            h = 0.5 * h * (1.0 + lax.erf(h * 0.7071067811865476))
            yc = jnp.dot(h, w2_ref[0, sl, :],
                         precision=lax.Precision.DEFAULT,
                         preferred_element_type=jnp.float32)
            y = yc if y is None else y + yc
        ys_ref[...] = y + b2_ref[0]


def _ffn(te, xs, w1b, b1, w2b, b2):
    grid_spec = pltpu.PrefetchScalarGridSpec(
        num_scalar_prefetch=1,
        grid=(NT,),
        in_specs=[
            pl.BlockSpec((T, DIM), lambda g, te_ref: (g, 0)),
            pl.BlockSpec((1, DIM, HID),
                         lambda g, te_ref: (jnp.minimum(te_ref[g], E - 1),
                                            0, 0)),
            pl.BlockSpec((1, 1, HID),
                         lambda g, te_ref: (jnp.minimum(te_ref[g], E - 1),
                                            0, 0)),
            pl.BlockSpec((1, HID, DIM),
                         lambda g, te_ref: (jnp.minimum(te_ref[g], E - 1),
                                            0, 0)),
            pl.BlockSpec((1, 1, DIM),
                         lambda g, te_ref: (jnp.minimum(te_ref[g], E - 1),
                                            0, 0)),
        ],
        out_specs=pl.BlockSpec((T, DIM), lambda g, te_ref: (g, 0)),
    )
    return pl.pallas_call(
        _ffn_body,
        grid_spec=grid_spec,
        out_shape=jax.ShapeDtypeStruct((M, DIM), jnp.float32),
        compiler_params=pltpu.CompilerParams(
            dimension_semantics=("arbitrary",),
            vmem_limit_bytes=120 * 1024 * 1024),
    )(te, xs, w1b, b1.reshape(E, 1, HID), w2b, b2.reshape(E, 1, DIM))


NCHC = TOK_W // CH2


def _combine_body(ys_hbm, pos0_hbm, pos1_hbm, g0_hbm, g1_hbm, out_hbm,
                  abuf, bbuf, obuf, p0v, p1v, g0v, g1v,
                  gsem0, gsem1, osem0, osem1):
    wid = lax.axis_index("s") * NC + lax.axis_index("c")
    t0 = wid * TOK_W
    lanes = lax.iota(jnp.int32, 16)
    gsems = (gsem0, gsem1)
    osems = (osem0, osem1)

    def gfire(ci):
        par = ci % 2
        off = t0 + ci * CH2
        pltpu.sync_copy(pos0_hbm.at[pl.ds(off, CH2)], p0v.at[par])
        pltpu.sync_copy(pos1_hbm.at[pl.ds(off, CH2)], p1v.at[par])
        pltpu.sync_copy(g0_hbm.at[pl.ds(off, CH2)], g0v.at[par])
        pltpu.sync_copy(g1_hbm.at[pl.ds(off, CH2)], g1v.at[par])
        return (pltpu.async_copy(ys_hbm.at[p0v.at[par]], abuf.at[par],
                                 gsems[par]),
                pltpu.async_copy(ys_hbm.at[p1v.at[par]], bbuf.at[par],
                                 gsems[par]))

    gats = {0: gfire(0)}
    osts = {}
    for ci in range(NCHC):
        par = ci % 2
        if ci + 1 < NCHC:
            gats[ci + 1] = gfire(ci + 1)
        gats[ci][0].wait()
        gats[ci][1].wait()
        if ci - 2 in osts:
            osts[ci - 2].wait()
            del osts[ci - 2]
        g0reg = g0v[par]
        g1reg = g1v[par]
        for i in range(CH2):
            iv = lanes * 0 + i
            s0 = _take16(g0reg, iv)
            s1 = _take16(g1reg, iv)

            def col(d, c3, par=par, i=i, s0=s0, s1=s1):
                for u in range(4):
                    cs = pl.ds((d * 4 + u) * 16, 16)
                    obuf[par, i, cs] = (abuf[par, i, cs] * s0
                                        + bbuf[par, i, cs] * s1)
                return c3

            lax.fori_loop(0, DIM // 64, col, 0)
        osts[ci] = pltpu.async_copy(obuf.at[par],
                                    out_hbm.at[pl.ds(t0 + ci * CH2, CH2)],
                                    osems[par])
    for o in osts.values():
        o.wait()


@functools.lru_cache(maxsize=None)
def _make_combine():
    mesh = plsc.VectorSubcoreMesh(core_axis_name="c", subcore_axis_name="s")
    return pl.kernel(
        _combine_body, mesh=mesh,
        out_type=jax.ShapeDtypeStruct((N, DIM), jnp.float32),
        scratch_types=[
            pltpu.VMEM((2, CH2, DIM), jnp.float32),
            pltpu.VMEM((2, CH2, DIM), jnp.float32),
            pltpu.VMEM((2, CH2, DIM), jnp.float32),
            pltpu.VMEM((2, CH2), jnp.int32),
            pltpu.VMEM((2, CH2), jnp.int32),
            pltpu.VMEM((2, CH2), jnp.float32),
            pltpu.VMEM((2, CH2), jnp.float32),
            pltpu.SemaphoreType.DMA,
            pltpu.SemaphoreType.DMA,
            pltpu.SemaphoreType.DMA,
            pltpu.SemaphoreType.DMA,
        ],
    )


def kernel(x, Wr, W1, b1, W2, b2):
    x2 = x.reshape(N, DIM)
    e0, e1, r0, r1, g0, g1, base16, te, aux = _router(x2, Wr)
    base16 = base16.reshape(E16)
    te = te.reshape(NTP)[:NT]

    xs, pos0, pos1 = _make_dispatch()(x2, e0.reshape(N), e1.reshape(N),
                                      r0.reshape(N), r1.reshape(N), base16)
    ys = _ffn(te, xs, W1.astype(jnp.bfloat16), b1, W2, b2)
    out2 = _make_combine()(ys, pos0, pos1, g0.reshape(N), g1.reshape(N))
    return out2.reshape(B, S, DIM), aux.reshape(())


# speedup vs baseline: 6.6439x; 1.0072x over previous
import functools

import jax
import jax.numpy as jnp
from jax import lax
from jax.experimental import pallas as pl
from jax.experimental.pallas import tpu as pltpu
from jax.experimental.pallas import tpu_sc as plsc

B, S, DIM, HID, E, K = 2, 2048, 1024, 4096, 8, 2
N = B * S
A = N * K
T = 256
M = A + E * T
NT = M // T
TB = 512
NTR = N // TB
E16 = 16

NC, NS = 2, 16
NW = NC * NS
TOK_W = N // NW
CH = 32
CH2 = 16


NTP = 128


def _router_body(x_ref, wr_ref, w1_ref, e0_ref, e1_ref, r0_ref, r1_ref,
                 g0_ref, g1_ref, base_ref, te_ref, aux_ref, w1b_ref,
                 cnt_acc, psum_acc):
    i = pl.program_id(0)
    w1b_ref[...] = w1_ref[...].astype(jnp.bfloat16)
    x = x_ref[...]
    logits = jnp.dot(x, wr_ref[...], preferred_element_type=jnp.float32)
    m = jnp.max(logits, axis=-1, keepdims=True)
    p = jnp.exp(logits - m)
    p = p / jnp.sum(p, axis=-1, keepdims=True)

    i0 = jnp.argmax(p, axis=-1)
    eidx = lax.broadcasted_iota(jnp.int32, (TB, E16), 1)
    oh0 = (eidx == i0[:, None]).astype(jnp.float32)
    pm = jnp.where(eidx[:, :E] == i0[:, None], -jnp.inf, p)
    i1 = jnp.argmax(pm, axis=-1)
    oh1 = (eidx == i1[:, None]).astype(jnp.float32)
    m0 = jnp.max(p, axis=-1)
    m1 = jnp.max(pm, axis=-1)
    den = m0 + m1 + 1e-9
    g0 = m0 / den
    g1 = m1 / den

    @pl.when(i == 0)
    def _():
        cnt_acc[...] = jnp.zeros_like(cnt_acc)
        psum_acc[...] = jnp.zeros_like(psum_acc)

    ohsum = oh0 + oh1
    rows = lax.broadcasted_iota(jnp.int32, (TB, TB), 0)
    cols = lax.broadcasted_iota(jnp.int32, (TB, TB), 1)
    ltri = (cols < rows).astype(jnp.float32)
    csum = jnp.dot(ltri, ohsum, preferred_element_type=jnp.float32)
    before = cnt_acc[...] + csum
    r0 = jnp.sum(before * oh0, axis=-1)
    r1 = jnp.sum(before * oh1, axis=-1)

    e0_ref[...] = i0.astype(jnp.int32)[:, None]
    e1_ref[...] = i1.astype(jnp.int32)[:, None]
    r0_ref[...] = r0.astype(jnp.int32)[:, None]
    r1_ref[...] = r1.astype(jnp.int32)[:, None]
    g0_ref[...] = g0[:, None]
    g1_ref[...] = g1[:, None]

    newcnt = cnt_acc[...] + jnp.sum(ohsum, axis=0, keepdims=True)
    cnt_acc[...] = newcnt
    newpsum = psum_acc[...] + jnp.sum(p, axis=0, keepdims=True)
    psum_acc[...] = newpsum

    @pl.when(i == NTR - 1)
    def _():
        avg_tokens = newcnt[0, :E] / float(N * K)
        avg_prob = newpsum[0] / float(N)
        aux_ref[...] = (0.01 * E * jnp.sum(avg_tokens * avg_prob,
                                           keepdims=True))[:, None]
        cnt_i = newcnt.astype(jnp.int32)
        pc = ((cnt_i + (T - 1)) // T) * T
        ut = (lax.broadcasted_iota(jnp.int32, (E16, E16), 0)
              <= lax.broadcasted_iota(jnp.int32, (E16, E16), 1))
        incl = jnp.dot(pc.astype(jnp.float32), ut.astype(jnp.float32),
                       preferred_element_type=jnp.float32).astype(jnp.int32)
        base_ref[...] = incl - pc
        gstart = lax.broadcasted_iota(jnp.int32, (NTP, E16), 0) * T
        eidx2 = lax.broadcasted_iota(jnp.int32, (NTP, E16), 1)
        ge = jnp.where(eidx2 < E, (gstart >= incl).astype(jnp.int32), 0)
        tev = jnp.minimum(jnp.sum(ge, axis=1), E - 1)
        totals = jnp.sum(jnp.where(eidx2 == E - 1, incl, 0), axis=1)
        te_ref[...] = jnp.where(gstart[:, 0] >= totals, E, tev)[:, None]


def _router(x2, Wr, W1):
    assert NTR == E
    iv = [jax.ShapeDtypeStruct((N, 1), jnp.int32)] * 4
    fv = [jax.ShapeDtypeStruct((N, 1), jnp.float32)] * 2
    return pl.pallas_call(
        _router_body,
        grid=(NTR,),
        in_specs=[
            pl.BlockSpec((TB, DIM), lambda i: (i, 0)),
            pl.BlockSpec((DIM, E), lambda i: (0, 0)),
            pl.BlockSpec((1, DIM, HID), lambda i: (i, 0, 0)),
        ],
        out_specs=[pl.BlockSpec((TB, 1), lambda i: (i, 0))] * 6
        + [pl.BlockSpec((1, E16), lambda i: (0, 0)),
           pl.BlockSpec((NTP, 1), lambda i: (0, 0)),
           pl.BlockSpec((1, 1), lambda i: (0, 0)),
           pl.BlockSpec((1, DIM, HID), lambda i: (i, 0, 0))],
        out_shape=iv + fv + [jax.ShapeDtypeStruct((1, E16), jnp.int32),
                             jax.ShapeDtypeStruct((NTP, 1), jnp.int32),
                             jax.ShapeDtypeStruct((1, 1), jnp.float32),
                             jax.ShapeDtypeStruct((E, DIM, HID),
                                                  jnp.bfloat16)],
        scratch_shapes=[pltpu.VMEM((1, E16), jnp.float32),
                        pltpu.VMEM((1, E), jnp.float32)],
        compiler_params=pltpu.CompilerParams(
            dimension_semantics=("arbitrary",),
            vmem_limit_bytes=120 * 1024 * 1024),
    )(x2, Wr, W1)


def _take16(vec, idx):
    return lax.gather(
        vec, idx[:, None],
        dimension_numbers=lax.GatherDimensionNumbers(
            offset_dims=(), collapsed_slice_dims=(0,), start_index_map=(0,)),
        slice_sizes=(1,),
        mode=lax.GatherScatterMode.PROMISE_IN_BOUNDS)

NCHD = TOK_W // CH
NBUF = 3


def _dispatch_body(x_hbm, e0_hbm, e1_hbm, r0_hbm, r1_hbm, base_hbm,
                   xs_hbm, pos0_hbm, pos1_hbm,
                   xbuf, basev, e0v, e1v, r0v, r1v, p0v, p1v,
                   xsem0, xsem1, xsem2, ssem0, ssem1, ssem2):
    wid = lax.axis_index("s") * NC + lax.axis_index("c")
    t0 = wid * TOK_W
    pltpu.sync_copy(base_hbm, basev)
    basereg = basev[...]
    xsems = (xsem0, xsem1, xsem2)
    ssems = (ssem0, ssem1, ssem2)

    def prep(ci):
        off = t0 + ci * CH
        pltpu.sync_copy(e0_hbm.at[pl.ds(off, CH)], e0v)
        pltpu.sync_copy(e1_hbm.at[pl.ds(off, CH)], e1v)
        pltpu.sync_copy(r0_hbm.at[pl.ds(off, CH)], r0v)
        pltpu.sync_copy(r1_hbm.at[pl.ds(off, CH)], r1v)
        for j in range(CH // 16):
            sl = pl.ds(j * 16, 16)
            p0v[ci, sl] = _take16(basereg, e0v[sl]) + r0v[sl]
            p1v[ci, sl] = _take16(basereg, e1v[sl]) + r1v[sl]
        pltpu.sync_copy(p0v.at[ci], pos0_hbm.at[pl.ds(off, CH)])
        pltpu.sync_copy(p1v.at[ci], pos1_hbm.at[pl.ds(off, CH)])
        return pltpu.async_copy(x_hbm.at[pl.ds(off, CH)],
                                xbuf.at[ci % NBUF], xsems[ci % NBUF])

    loads = {0: prep(0), 1: prep(1)}
    scats = {}
    for ci in range(NCHD):
        b = ci % NBUF
        loads[ci].wait()
        scats[ci] = (
            pltpu.async_copy(xbuf.at[b], xs_hbm.at[p0v.at[ci]], ssems[b]),
            pltpu.async_copy(xbuf.at[b], xs_hbm.at[p1v.at[ci]], ssems[b]),
        )
        nxt = ci + 2
        if nxt < NCHD:
            if nxt - NBUF in scats:
                scats[nxt - NBUF][0].wait()
                scats[nxt - NBUF][1].wait()
                del scats[nxt - NBUF]
            loads[nxt] = prep(nxt)
    for pair in scats.values():
        pair[0].wait()
        pair[1].wait()


@functools.lru_cache(maxsize=None)
def _make_dispatch():
    mesh = plsc.VectorSubcoreMesh(core_axis_name="c", subcore_axis_name="s")
    return pl.kernel(
        _dispatch_body, mesh=mesh,
        out_type=[
            jax.ShapeDtypeStruct((M, DIM), jnp.float32),
            jax.ShapeDtypeStruct((N,), jnp.int32),
            jax.ShapeDtypeStruct((N,), jnp.int32),
        ],
        scratch_types=[
            pltpu.VMEM((NBUF, CH, DIM), jnp.float32),
            pltpu.VMEM((E16,), jnp.int32),
            pltpu.VMEM((CH,), jnp.int32),
            pltpu.VMEM((CH,), jnp.int32),
            pltpu.VMEM((CH,), jnp.int32),
            pltpu.VMEM((CH,), jnp.int32),
            pltpu.VMEM((NCHD, CH), jnp.int32),
            pltpu.VMEM((NCHD, CH), jnp.int32),
            pltpu.SemaphoreType.DMA,
            pltpu.SemaphoreType.DMA,
            pltpu.SemaphoreType.DMA,
            pltpu.SemaphoreType.DMA,
            pltpu.SemaphoreType.DMA,
            pltpu.SemaphoreType.DMA,
        ],
    )


CHID = 1024


def _ffn_body(te_ref, xs_ref, w1_ref, b1_ref, w2_ref, b2_ref, ys_ref):
    g = pl.program_id(0)

    @pl.when(te_ref[g] < E)
    def _():
        xb = xs_ref[...].astype(jnp.bfloat16)
        y = None
        for c in range(HID // CHID):
            sl = pl.ds(c * CHID, CHID)
            h = jnp.dot(xb, w1_ref[0, :, sl],
                        preferred_element_type=jnp.float32)
            h = h + b1_ref[0, :, sl]
            h = 0.5 * h * (1.0 + lax.erf(h * 0.7071067811865476))
            yc = jnp.dot(h, w2_ref[0, sl, :],
                         precision=lax.Precision.DEFAULT,
                         preferred_element_type=jnp.float32)
            y = yc if y is None else y + yc
        ys_ref[...] = y + b2_ref[0]


def _ffn(te, xs, w1b, b1, w2b, b2):
    grid_spec = pltpu.PrefetchScalarGridSpec(
        num_scalar_prefetch=1,
        grid=(NT,),
        in_specs=[
            pl.BlockSpec((T, DIM), lambda g, te_ref: (g, 0)),
            pl.BlockSpec((1, DIM, HID),
                         lambda g, te_ref: (jnp.minimum(te_ref[g], E - 1),
                                            0, 0)),
            pl.BlockSpec((1, 1, HID),
                         lambda g, te_ref: (jnp.minimum(te_ref[g], E - 1),
                                            0, 0)),
            pl.BlockSpec((1, HID, DIM),
                         lambda g, te_ref: (jnp.minimum(te_ref[g], E - 1),
                                            0, 0)),
            pl.BlockSpec((1, 1, DIM),
                         lambda g, te_ref: (jnp.minimum(te_ref[g], E - 1),
                                            0, 0)),
        ],
        out_specs=pl.BlockSpec((T, DIM), lambda g, te_ref: (g, 0)),
    )
    return pl.pallas_call(
        _ffn_body,
        grid_spec=grid_spec,
        out_shape=jax.ShapeDtypeStruct((M, DIM), jnp.float32),
        compiler_params=pltpu.CompilerParams(
            dimension_semantics=("arbitrary",),
            vmem_limit_bytes=120 * 1024 * 1024),
    )(te, xs, w1b, b1.reshape(E, 1, HID), w2b, b2.reshape(E, 1, DIM))


NCHC = TOK_W // CH2


def _combine_body(ys_hbm, pos0_hbm, pos1_hbm, g0_hbm, g1_hbm, out_hbm,
                  abuf, bbuf, obuf, p0v, p1v, g0v, g1v,
                  gsem0, gsem1, osem0, osem1):
    wid = lax.axis_index("s") * NC + lax.axis_index("c")
    t0 = wid * TOK_W
    lanes = lax.iota(jnp.int32, 16)
    gsems = (gsem0, gsem1)
    osems = (osem0, osem1)

    def gfire(ci):
        par = ci % 2
        off = t0 + ci * CH2
        pltpu.sync_copy(pos0_hbm.at[pl.ds(off, CH2)], p0v.at[par])
        pltpu.sync_copy(pos1_hbm.at[pl.ds(off, CH2)], p1v.at[par])
        pltpu.sync_copy(g0_hbm.at[pl.ds(off, CH2)], g0v.at[par])
        pltpu.sync_copy(g1_hbm.at[pl.ds(off, CH2)], g1v.at[par])
        return (pltpu.async_copy(ys_hbm.at[p0v.at[par]], abuf.at[par],
                                 gsems[par]),
                pltpu.async_copy(ys_hbm.at[p1v.at[par]], bbuf.at[par],
                                 gsems[par]))

    gats = {0: gfire(0)}
    osts = {}
    for ci in range(NCHC):
        par = ci % 2
        if ci + 1 < NCHC:
            gats[ci + 1] = gfire(ci + 1)
        gats[ci][0].wait()
        gats[ci][1].wait()
        if ci - 2 in osts:
            osts[ci - 2].wait()
            del osts[ci - 2]
        g0reg = g0v[par]
        g1reg = g1v[par]
        for i in range(CH2):
            iv = lanes * 0 + i
            s0 = _take16(g0reg, iv)
            s1 = _take16(g1reg, iv)

            def col(d, c3, par=par, i=i, s0=s0, s1=s1):
                for u in range(4):
                    cs = pl.ds((d * 4 + u) * 16, 16)
                    obuf[par, i, cs] = (abuf[par, i, cs] * s0
                                        + bbuf[par, i, cs] * s1)
                return c3

            lax.fori_loop(0, DIM // 64, col, 0)
        osts[ci] = pltpu.async_copy(obuf.at[par],
                                    out_hbm.at[pl.ds(t0 + ci * CH2, CH2)],
                                    osems[par])
    for o in osts.values():
        o.wait()


@functools.lru_cache(maxsize=None)
def _make_combine():
    mesh = plsc.VectorSubcoreMesh(core_axis_name="c", subcore_axis_name="s")
    return pl.kernel(
        _combine_body, mesh=mesh,
        out_type=jax.ShapeDtypeStruct((N, DIM), jnp.float32),
        scratch_types=[
            pltpu.VMEM((2, CH2, DIM), jnp.float32),
            pltpu.VMEM((2, CH2, DIM), jnp.float32),
            pltpu.VMEM((2, CH2, DIM), jnp.float32),
            pltpu.VMEM((2, CH2), jnp.int32),
            pltpu.VMEM((2, CH2), jnp.int32),
            pltpu.VMEM((2, CH2), jnp.float32),
            pltpu.VMEM((2, CH2), jnp.float32),
            pltpu.SemaphoreType.DMA,
            pltpu.SemaphoreType.DMA,
            pltpu.SemaphoreType.DMA,
            pltpu.SemaphoreType.DMA,
        ],
    )


def kernel(x, Wr, W1, b1, W2, b2):
    x2 = x.reshape(N, DIM)
    e0, e1, r0, r1, g0, g1, base16, te, aux, w1b = _router(x2, Wr, W1)
    base16 = base16.reshape(E16)
    te = te.reshape(NTP)[:NT]

    xs, pos0, pos1 = _make_dispatch()(x2, e0.reshape(N), e1.reshape(N),
                                      r0.reshape(N), r1.reshape(N), base16)
    ys = _ffn(te, xs, w1b, b1, W2, b2)
    out2 = _make_combine()(ys, pos0, pos1, g0.reshape(N), g1.reshape(N))
    return out2.reshape(B, S, DIM), aux.reshape(())
